# Initial kernel scaffold; baseline (speedup 1.0000x reference)
import jax
import jax.numpy as jnp
from jax.experimental import pallas as pl


def kernel(z, pos, batch, edge_index, emb, mw1_0, mb1_0, mw2_0, mb2_0, l1w_0, l2w_0, l2b_0, lw_0, lb_0, mw1_1, mb1_1, mw2_1, mb2_1, l1w_1, l2w_1, l2b_1, lw_1, lb_1, fl1w, fl1b, fl2w, fl2b, pw, pb):
    raise NotImplementedError("write your pallas kernel here")


# speedup vs baseline: 1.8134x; 1.8134x over previous
import functools

import jax
import jax.numpy as jnp
import numpy as np
from jax import lax
from jax.experimental import pallas as pl
from jax.experimental.pallas import tpu as pltpu
from jax.experimental.pallas import tpu_sc as plsc

N = 10000
E = 320000
H = 128
NGAUSS = 10
NGRAPH = 64
CUTOFF = 10.0

NW = 32
EPT = 10240
E_PAD = EPT * NW
CH = 128
NPT = 320
N_PAD = NPT * NW
BE = 1024
BN = 1024

_LOG2 = float(np.log(2.0))
_DELTA = CUTOFF / (NGAUSS - 1)
_COEFF = -0.5 / (_DELTA * _DELTA)
_OFFS = np.full((1, 16), 1e4, np.float32)
_OFFS[0, :NGAUSS] = np.linspace(0.0, CUTOFF, NGAUSS, dtype=np.float32)


def _ssp(x):
    return jnp.maximum(x, 0.0) + jnp.log1p(jnp.exp(-jnp.abs(x))) - _LOG2


def _sc_prep_body(px_hbm, py_hbm, pz_hbm, src_hbm, dst_hbm, z_hbm, emb_hbm,
                  d2_out, h0_out,
                  src_v, dst_v, sx_v, sy_v, sz_v, tx_v, ty_v, tz_v,
                  d2_v, z_v, emb_v, sem):
    cid = lax.axis_index("c")
    sid = lax.axis_index("s")
    wid = cid * 16 + sid
    ebase = wid * EPT
    pltpu.sync_copy(src_hbm.at[pl.ds(ebase, EPT)], src_v)
    pltpu.sync_copy(dst_hbm.at[pl.ds(ebase, EPT)], dst_v)

    def chunk(c, carry):
        isrc = src_v.at[pl.ds(c * CH, CH)]
        idst = dst_v.at[pl.ds(c * CH, CH)]
        cps = [pltpu.async_copy(px_hbm.at[isrc], sx_v, sem),
               pltpu.async_copy(py_hbm.at[isrc], sy_v, sem),
               pltpu.async_copy(pz_hbm.at[isrc], sz_v, sem),
               pltpu.async_copy(px_hbm.at[idst], tx_v, sem),
               pltpu.async_copy(py_hbm.at[idst], ty_v, sem),
               pltpu.async_copy(pz_hbm.at[idst], tz_v, sem)]
        for cp in cps:
            cp.wait()
        for v in range(CH // 16):
            sl = pl.ds(v * 16, 16)
            ddx = sx_v[sl] - tx_v[sl]
            ddy = sy_v[sl] - ty_v[sl]
            ddz = sz_v[sl] - tz_v[sl]
            d2_v[pl.ds(c * CH + v * 16, 16)] = (
                ddx * ddx + ddy * ddy + ddz * ddz)
        return carry

    lax.fori_loop(0, EPT // CH, chunk, 0)
    pltpu.sync_copy(d2_v, d2_out.at[pl.ds(ebase, EPT)])

    nbase = wid * NPT
    pltpu.sync_copy(z_hbm.at[pl.ds(nbase, NPT)], z_v)
    for c0 in range(0, NPT, 80):
        pltpu.async_copy(emb_hbm.at[z_v.at[pl.ds(c0, 80)]],
                         emb_v.at[pl.ds(c0, 80)], sem).wait()
    pltpu.sync_copy(emb_v, h0_out.at[pl.ds(nbase, NPT)])


_sc_prep = functools.partial(
    pl.kernel,
    out_type=[jax.ShapeDtypeStruct((E_PAD,), jnp.float32),
              jax.ShapeDtypeStruct((N_PAD, H), jnp.float32)],
    mesh=plsc.VectorSubcoreMesh(core_axis_name="c", subcore_axis_name="s"),
    scratch_types=(
        [pltpu.VMEM((EPT,), jnp.int32),
         pltpu.VMEM((EPT,), jnp.int32)]
        + [pltpu.VMEM((CH,), jnp.float32) for _ in range(6)]
        + [pltpu.VMEM((EPT,), jnp.float32),
           pltpu.VMEM((NPT,), jnp.int32),
           pltpu.VMEM((NPT, H), jnp.float32),
           pltpu.SemaphoreType.DMA]
    ),
)(_sc_prep_body)


def _sc_msgpass_body(xl_hbm, wf_hbm, src_hbm, dst_hbm, zeros_hbm,
                     agg_out,
                     src_v, dst_v, xr_v, wf_v, acc_sh, sem):
    cid = lax.axis_index("c")
    sid = lax.axis_index("s")
    wid = cid * 16 + sid
    rpt = N_PAD // 16
    r0 = sid * rpt
    pltpu.sync_copy(zeros_hbm.at[pl.ds(r0, rpt)], acc_sh.at[pl.ds(r0, rpt)])
    plsc.subcore_barrier()

    ebase = wid * EPT

    def chunk(ch, carry):
        off = ebase + ch * CH
        pltpu.sync_copy(src_hbm.at[pl.ds(off, CH)], src_v)
        pltpu.sync_copy(dst_hbm.at[pl.ds(off, CH)], dst_v)
        gat = pltpu.async_copy(xl_hbm.at[src_v], xr_v, sem)
        pltpu.sync_copy(wf_hbm.at[pl.ds(off, CH)], wf_v)
        gat.wait()

        def row(r, c2):
            for v in range(8):
                sl = pl.ds(v * 16, 16)
                xr_v[r, sl] = xr_v[r, sl] * wf_v[r, sl]
            return c2

        lax.fori_loop(0, CH, row, 0)
        pltpu.sync_copy(xr_v, acc_sh.at[dst_v], add=True)
        return carry

    lax.fori_loop(0, EPT // CH, chunk, 0)
    plsc.subcore_barrier()
    pltpu.sync_copy(acc_sh.at[pl.ds(r0, rpt)],
                    agg_out.at[cid].at[pl.ds(r0, rpt)])


_sc_msgpass = functools.partial(
    pl.kernel,
    out_type=jax.ShapeDtypeStruct((2, N_PAD, H), jnp.float32),
    mesh=plsc.VectorSubcoreMesh(core_axis_name="c", subcore_axis_name="s"),
    scratch_types=[
        pltpu.VMEM((CH,), jnp.int32),
        pltpu.VMEM((CH,), jnp.int32),
        pltpu.VMEM((CH, H), jnp.float32),
        pltpu.VMEM((CH, H), jnp.float32),
        pltpu.VMEM_SHARED((N_PAD, H), jnp.float32),
        pltpu.SemaphoreType.DMA,
    ],
)(_sc_msgpass_body)


def _tc_wf_body(d2_ref, mw1p0, mb10, mw20, mb20, mw1p1, mb11, mw21, mb21,
                wf0_ref, wf1_ref):
    d2 = d2_ref[...]
    dist = jnp.sqrt(d2 + 1e-12)
    lane = lax.broadcasted_iota(jnp.int32, (1, 16), 1)
    offs = jnp.where(lane < NGAUSS, lane.astype(jnp.float32) * _DELTA, 1e4)
    diff = dist - offs
    rbf = jnp.exp(_COEFF * (diff * diff))
    c = 0.5 * (jnp.cos(dist * (np.pi / CUTOFF)) + 1.0)
    rows = pl.program_id(0) * BE + lax.broadcasted_iota(jnp.int32, (BE, 1), 0)
    c = jnp.where(rows < E, c, 0.0)
    for (w1, b1, w2, b2, out) in ((mw1p0, mb10, mw20, mb20, wf0_ref),
                                  (mw1p1, mb11, mw21, mb21, wf1_ref)):
        t = _ssp(jnp.dot(rbf, w1[...], preferred_element_type=jnp.float32)
                 + b1[...])
        out[...] = (jnp.dot(t, w2[...], preferred_element_type=jnp.float32)
                    + b2[...]) * c


def _tc_wf(d2, mw1p0, mb10, mw20, mb20, mw1p1, mb11, mw21, mb21):
    g = E_PAD // BE
    wspec = lambda shp: pl.BlockSpec(shp, lambda i: (0, 0))
    return pl.pallas_call(
        _tc_wf_body,
        grid=(g,),
        in_specs=[
            pl.BlockSpec((BE, 1), lambda i: (i, 0)),
            wspec((16, H)), wspec((1, H)), wspec((H, H)), wspec((1, H)),
            wspec((16, H)), wspec((1, H)), wspec((H, H)), wspec((1, H)),
        ],
        out_specs=[pl.BlockSpec((BE, H), lambda i: (i, 0)),
                   pl.BlockSpec((BE, H), lambda i: (i, 0))],
        out_shape=[jax.ShapeDtypeStruct((E_PAD, H), jnp.float32),
                   jax.ShapeDtypeStruct((E_PAD, H), jnp.float32)],
    )(d2, mw1p0, mb10, mw20, mb20, mw1p1, mb11, mw21, mb21)


def _tc_xl_body(h_ref, w_ref, o_ref):
    o_ref[...] = jnp.dot(h_ref[...], w_ref[...],
                         preferred_element_type=jnp.float32)


def _tc_xl(h, w):
    return pl.pallas_call(
        _tc_xl_body,
        grid=(N_PAD // BN,),
        in_specs=[pl.BlockSpec((BN, H), lambda i: (i, 0)),
                  pl.BlockSpec((H, H), lambda i: (0, 0))],
        out_specs=pl.BlockSpec((BN, H), lambda i: (i, 0)),
        out_shape=jax.ShapeDtypeStruct((N_PAD, H), jnp.float32),
    )(h, w)


def _tc_update_body(p0_ref, p1_ref, h_ref, l2w, l2b, lw, lb, l1wn,
                    hn_ref, xln_ref):
    agg = p0_ref[...] + p1_ref[...]
    t = _ssp(jnp.dot(agg, l2w[...], preferred_element_type=jnp.float32)
             + l2b[...])
    y = jnp.dot(t, lw[...], preferred_element_type=jnp.float32) + lb[...]
    hn = h_ref[...] + y
    hn_ref[...] = hn
    xln_ref[...] = jnp.dot(hn, l1wn[...], preferred_element_type=jnp.float32)


def _tc_update(p0, p1, h, l2w, l2b, lw, lb, l1wn):
    wspec = lambda shp: pl.BlockSpec(shp, lambda i: (0, 0))
    return pl.pallas_call(
        _tc_update_body,
        grid=(N_PAD // BN,),
        in_specs=[pl.BlockSpec((BN, H), lambda i: (i, 0)),
                  pl.BlockSpec((BN, H), lambda i: (i, 0)),
                  pl.BlockSpec((BN, H), lambda i: (i, 0)),
                  wspec((H, H)), wspec((1, H)), wspec((H, H)), wspec((1, H)),
                  wspec((H, H))],
        out_specs=[pl.BlockSpec((BN, H), lambda i: (i, 0)),
                   pl.BlockSpec((BN, H), lambda i: (i, 0))],
        out_shape=[jax.ShapeDtypeStruct((N_PAD, H), jnp.float32),
                   jax.ShapeDtypeStruct((N_PAD, H), jnp.float32)],
    )(p0, p1, h, l2w, l2b, lw, lb, l1wn)


def _tc_readout_body(h2p_ref, fl1b, fl2w, fl2b, batch_ref, pw, pb,
                     out_ref, g_ref):
    t = _ssp(h2p_ref[...] + fl1b[...])
    h2 = jnp.dot(t, fl2w[...], preferred_element_type=jnp.float32) + fl2b[...]
    bt = batch_ref[...]
    oh = (bt == lax.broadcasted_iota(jnp.int32, (1, NGRAPH), 1))
    oh = oh.astype(jnp.float32)
    g_part = lax.dot_general(oh, h2, (((0,), (0,)), ((), ())),
                             preferred_element_type=jnp.float32)
    pid = pl.program_id(0)

    @pl.when(pid == 0)
    def _():
        g_ref[...] = g_part

    @pl.when(pid > 0)
    def _():
        g_ref[...] = g_ref[...] + g_part

    @pl.when(pid == pl.num_programs(0) - 1)
    def _():
        out_ref[...] = (jnp.dot(g_ref[...], pw[...],
                                preferred_element_type=jnp.float32) + pb[...])


def _tc_readout(h2p, fl1b, fl2w, fl2b, batch2d, pw, pb):
    wspec = lambda shp: pl.BlockSpec(shp, lambda i: (0, 0))
    return pl.pallas_call(
        _tc_readout_body,
        grid=(N_PAD // BN,),
        in_specs=[pl.BlockSpec((BN, H), lambda i: (i, 0)),
                  wspec((1, H)), wspec((H, H)), wspec((1, H)),
                  pl.BlockSpec((BN, 1), lambda i: (i, 0)),
                  wspec((H, 1)), wspec((1, 1))],
        out_specs=pl.BlockSpec((NGRAPH, 1), lambda i: (0, 0)),
        out_shape=jax.ShapeDtypeStruct((NGRAPH, 1), jnp.float32),
        scratch_shapes=[pltpu.VMEM((NGRAPH, H), jnp.float32)],
    )(h2p, fl1b, fl2w, fl2b, batch2d, pw, pb)


def kernel(z, pos, batch, edge_index, emb, mw1_0, mb1_0, mw2_0, mb2_0, l1w_0,
           l2w_0, l2b_0, lw_0, lb_0, mw1_1, mb1_1, mw2_1, mb2_1, l1w_1,
           l2w_1, l2b_1, lw_1, lb_1, fl1w, fl1b, fl2w, fl2b, pw, pb):
    src = edge_index[0].astype(jnp.int32)
    dst = edge_index[1].astype(jnp.int32)
    epad = E_PAD - E
    src_p = jnp.concatenate([src, jnp.zeros((epad,), jnp.int32)])
    dst_p = jnp.concatenate([dst, jnp.zeros((epad,), jnp.int32)])
    z_p = jnp.concatenate([z.astype(jnp.int32),
                           jnp.zeros((N_PAD - N,), jnp.int32)])
    batch_p = jnp.concatenate([batch.astype(jnp.int32),
                               jnp.full((N_PAD - N,), NGRAPH, jnp.int32)])
    zeros_tbl = jnp.zeros((N_PAD, H), jnp.float32)

    mw1p0 = jnp.zeros((16, H), jnp.float32).at[:NGAUSS].set(mw1_0)
    mw1p1 = jnp.zeros((16, H), jnp.float32).at[:NGAUSS].set(mw1_1)
    fl1wp = jnp.zeros((H, H), jnp.float32).at[:, :H // 2].set(fl1w)
    fl1bp = jnp.zeros((1, H), jnp.float32).at[0, :H // 2].set(fl1b)
    fl2wp = jnp.zeros((H, H), jnp.float32).at[:H // 2].set(fl2w)
    r2 = lambda v: v.reshape(1, -1)

    dist2, h0 = _sc_prep(pos[:, 0], pos[:, 1], pos[:, 2], src_p, dst_p,
                         z_p, emb)
    wf0, wf1 = _tc_wf(dist2.reshape(E_PAD, 1), mw1p0, r2(mb1_0), mw2_0,
                      r2(mb2_0), mw1p1, r2(mb1_1), mw2_1, r2(mb2_1))

    xl0 = _tc_xl(h0, l1w_0)
    agg0 = _sc_msgpass(xl0, wf0, src_p, dst_p, zeros_tbl)
    h1, xl1 = _tc_update(agg0[0], agg0[1], h0, l2w_0, r2(l2b_0), lw_0,
                         r2(lb_0), l1w_1)

    agg1 = _sc_msgpass(xl1, wf1, src_p, dst_p, zeros_tbl)
    h2, h2p = _tc_update(agg1[0], agg1[1], h1, l2w_1, r2(l2b_1), lw_1,
                         r2(lb_1), fl1wp)
    del h2

    out = _tc_readout(h2p, fl1bp, fl2wp, r2(fl2b), batch_p.reshape(N_PAD, 1),
                      pw, r2(pb))
    return out


# speedup vs baseline: 1.9222x; 1.0600x over previous
import functools

import jax
import jax.numpy as jnp
import numpy as np
from jax import lax
from jax.experimental import pallas as pl
from jax.experimental.pallas import tpu as pltpu
from jax.experimental.pallas import tpu_sc as plsc

N = 10000
E = 320000
H = 128
NGAUSS = 10
NGRAPH = 64
CUTOFF = 10.0

NW = 32
EPT = 10240
E_PAD = EPT * NW
CH = 128
NPT = 320
N_PAD = NPT * NW
BE = 1024
BN = 1024

_LOG2 = float(np.log(2.0))
_DELTA = CUTOFF / (NGAUSS - 1)
_COEFF = -0.5 / (_DELTA * _DELTA)
_OFFS = np.full((1, 16), 1e4, np.float32)
_OFFS[0, :NGAUSS] = np.linspace(0.0, CUTOFF, NGAUSS, dtype=np.float32)


def _ssp(x):
    return jnp.maximum(x, 0.0) + jnp.log1p(jnp.exp(-jnp.abs(x))) - _LOG2


def _sc_prep_body(px_hbm, py_hbm, pz_hbm, src_hbm, dst_hbm, z_hbm, emb_hbm,
                  d2_out, h0_out,
                  src_v, dst_v, sx_v, sy_v, sz_v, tx_v, ty_v, tz_v,
                  ux_v, uy_v, uz_v, vx_v, vy_v, vz_v,
                  d2_v, z_v, emb_v, sem, sem2):
    cid = lax.axis_index("c")
    sid = lax.axis_index("s")
    wid = cid * 16 + sid
    ebase = wid * EPT
    pltpu.sync_copy(src_hbm.at[pl.ds(ebase, EPT)], src_v)
    pltpu.sync_copy(dst_hbm.at[pl.ds(ebase, EPT)], dst_v)

    def fire(c, bufs, sem_c):
        isrc = src_v.at[pl.ds(c * CH, CH)]
        idst = dst_v.at[pl.ds(c * CH, CH)]
        return [pltpu.async_copy(px_hbm.at[isrc], bufs[0], sem_c),
                pltpu.async_copy(py_hbm.at[isrc], bufs[1], sem_c),
                pltpu.async_copy(pz_hbm.at[isrc], bufs[2], sem_c),
                pltpu.async_copy(px_hbm.at[idst], bufs[3], sem_c),
                pltpu.async_copy(py_hbm.at[idst], bufs[4], sem_c),
                pltpu.async_copy(pz_hbm.at[idst], bufs[5], sem_c)]

    bufs_a = (sx_v, sy_v, sz_v, tx_v, ty_v, tz_v)
    bufs_b = (ux_v, uy_v, uz_v, vx_v, vy_v, vz_v)

    def pair(k, carry):
        cps_a = fire(2 * k, bufs_a, sem)
        cps_b = fire(2 * k + 1, bufs_b, sem2)
        for cp in cps_a:
            cp.wait()
        for c, bufs in ((2 * k, bufs_a), (2 * k + 1, bufs_b)):
            if bufs is bufs_b:
                for cp in cps_b:
                    cp.wait()
            for v in range(CH // 16):
                sl = pl.ds(v * 16, 16)
                ddx = bufs[0][sl] - bufs[3][sl]
                ddy = bufs[1][sl] - bufs[4][sl]
                ddz = bufs[2][sl] - bufs[5][sl]
                d2_v[pl.ds(c * CH + v * 16, 16)] = (
                    ddx * ddx + ddy * ddy + ddz * ddz)
        return carry

    lax.fori_loop(0, EPT // CH // 2, pair, 0)
    pltpu.sync_copy(d2_v, d2_out.at[pl.ds(ebase, EPT)])

    nbase = wid * NPT
    pltpu.sync_copy(z_hbm.at[pl.ds(nbase, NPT)], z_v)
    for c0 in range(0, NPT, 80):
        pltpu.async_copy(emb_hbm.at[z_v.at[pl.ds(c0, 80)]],
                         emb_v.at[pl.ds(c0, 80)], sem).wait()
    pltpu.sync_copy(emb_v, h0_out.at[pl.ds(nbase, NPT)])


_sc_prep = functools.partial(
    pl.kernel,
    out_type=[jax.ShapeDtypeStruct((E_PAD,), jnp.float32),
              jax.ShapeDtypeStruct((N_PAD, H), jnp.float32)],
    mesh=plsc.VectorSubcoreMesh(core_axis_name="c", subcore_axis_name="s"),
    scratch_types=(
        [pltpu.VMEM((EPT,), jnp.int32),
         pltpu.VMEM((EPT,), jnp.int32)]
        + [pltpu.VMEM((CH,), jnp.float32) for _ in range(12)]
        + [pltpu.VMEM((EPT,), jnp.float32),
           pltpu.VMEM((NPT,), jnp.int32),
           pltpu.VMEM((NPT, H), jnp.float32),
           pltpu.SemaphoreType.DMA,
           pltpu.SemaphoreType.DMA]
    ),
)(_sc_prep_body)


CHM = 64


def _sc_msgpass_body(xl_hbm, wf_hbm, src_hbm, dst_hbm, zeros_hbm,
                     agg_out,
                     src_v, dc0, dc1, xr0, xr1, wf0, wf1, acc_sh,
                     sg0, sg1, sw0, sw1, ss0, ss1):
    cid = lax.axis_index("c")
    sid = lax.axis_index("s")
    wid = cid * 16 + sid
    rpt = N_PAD // 16
    r0 = sid * rpt
    pltpu.sync_copy(zeros_hbm.at[pl.ds(r0, rpt)], acc_sh.at[pl.ds(r0, rpt)])
    ebase = wid * EPT
    nch = EPT // CHM
    pltpu.sync_copy(src_hbm.at[pl.ds(ebase, EPT)], src_v)
    plsc.subcore_barrier()

    def mul(xr, wf):
        def row(r, c2):
            for v in range(8):
                sl = pl.ds(v * 16, 16)
                xr[r, sl] = xr[r, sl] * wf[r, sl]
            return c2

        lax.fori_loop(0, CHM, row, 0)

    def pair(k, carry):
        ch0 = 2 * k
        ch1 = 2 * k + 1
        g0 = pltpu.async_copy(xl_hbm.at[src_v.at[pl.ds(ch0 * CHM, CHM)]],
                              xr0, sg0)
        w0 = pltpu.async_copy(wf_hbm.at[pl.ds(ebase + ch0 * CHM, CHM)],
                              wf0, sw0)
        g1 = pltpu.async_copy(xl_hbm.at[src_v.at[pl.ds(ch1 * CHM, CHM)]],
                              xr1, sg1)
        w1 = pltpu.async_copy(wf_hbm.at[pl.ds(ch1 * CHM + ebase, CHM)],
                              wf1, sw1)
        pltpu.sync_copy(dst_hbm.at[pl.ds(ebase + ch0 * CHM, CHM)], dc0)
        pltpu.sync_copy(dst_hbm.at[pl.ds(ebase + ch1 * CHM, CHM)], dc1)
        g0.wait()
        w0.wait()
        mul(xr0, wf0)
        s0 = pltpu.async_copy(xr0, acc_sh.at[dc0], ss0, add=True)
        g1.wait()
        w1.wait()
        mul(xr1, wf1)
        s1 = pltpu.async_copy(xr1, acc_sh.at[dc1], ss1, add=True)
        s0.wait()
        s1.wait()
        return carry

    lax.fori_loop(0, nch // 2, pair, 0)
    plsc.subcore_barrier()
    pltpu.sync_copy(acc_sh.at[pl.ds(r0, rpt)],
                    agg_out.at[cid].at[pl.ds(r0, rpt)])


_sc_msgpass = functools.partial(
    pl.kernel,
    out_type=jax.ShapeDtypeStruct((2, N_PAD, H), jnp.float32),
    mesh=plsc.VectorSubcoreMesh(core_axis_name="c", subcore_axis_name="s"),
    scratch_types=[
        pltpu.VMEM((EPT,), jnp.int32),
        pltpu.VMEM((CHM,), jnp.int32),
        pltpu.VMEM((CHM,), jnp.int32),
        pltpu.VMEM((CHM, H), jnp.float32),
        pltpu.VMEM((CHM, H), jnp.float32),
        pltpu.VMEM((CHM, H), jnp.float32),
        pltpu.VMEM((CHM, H), jnp.float32),
        pltpu.VMEM_SHARED((N_PAD, H), jnp.float32),
        pltpu.SemaphoreType.DMA,
        pltpu.SemaphoreType.DMA,
        pltpu.SemaphoreType.DMA,
        pltpu.SemaphoreType.DMA,
        pltpu.SemaphoreType.DMA,
        pltpu.SemaphoreType.DMA,
    ],
)(_sc_msgpass_body)


def _tc_wf_body(d2_ref, mw1p0, mb10, mw20, mb20, mw1p1, mb11, mw21, mb21,
                wf0_ref, wf1_ref):
    d2 = d2_ref[...]
    dist = jnp.sqrt(d2 + 1e-12)
    lane = lax.broadcasted_iota(jnp.int32, (1, 16), 1)
    offs = jnp.where(lane < NGAUSS, lane.astype(jnp.float32) * _DELTA, 1e4)
    diff = dist - offs
    rbf = jnp.exp(_COEFF * (diff * diff))
    c = 0.5 * (jnp.cos(dist * (np.pi / CUTOFF)) + 1.0)
    rows = pl.program_id(0) * BE + lax.broadcasted_iota(jnp.int32, (BE, 1), 0)
    c = jnp.where(rows < E, c, 0.0)
    for (w1, b1, w2, b2, out) in ((mw1p0, mb10, mw20, mb20, wf0_ref),
                                  (mw1p1, mb11, mw21, mb21, wf1_ref)):
        t = _ssp(jnp.dot(rbf, w1[...], preferred_element_type=jnp.float32)
                 + b1[...])
        out[...] = (jnp.dot(t, w2[...], preferred_element_type=jnp.float32)
                    + b2[...]) * c


def _tc_wf(d2, mw1p0, mb10, mw20, mb20, mw1p1, mb11, mw21, mb21):
    g = E_PAD // BE
    wspec = lambda shp: pl.BlockSpec(shp, lambda i: (0, 0))
    return pl.pallas_call(
        _tc_wf_body,
        grid=(g,),
        in_specs=[
            pl.BlockSpec((BE, 1), lambda i: (i, 0)),
            wspec((16, H)), wspec((1, H)), wspec((H, H)), wspec((1, H)),
            wspec((16, H)), wspec((1, H)), wspec((H, H)), wspec((1, H)),
        ],
        out_specs=[pl.BlockSpec((BE, H), lambda i: (i, 0)),
                   pl.BlockSpec((BE, H), lambda i: (i, 0))],
        out_shape=[jax.ShapeDtypeStruct((E_PAD, H), jnp.float32),
                   jax.ShapeDtypeStruct((E_PAD, H), jnp.float32)],
    )(d2, mw1p0, mb10, mw20, mb20, mw1p1, mb11, mw21, mb21)


def _tc_xl_body(h_ref, w_ref, o_ref):
    o_ref[...] = jnp.dot(h_ref[...], w_ref[...],
                         preferred_element_type=jnp.float32)


def _tc_xl(h, w):
    return pl.pallas_call(
        _tc_xl_body,
        grid=(N_PAD // BN,),
        in_specs=[pl.BlockSpec((BN, H), lambda i: (i, 0)),
                  pl.BlockSpec((H, H), lambda i: (0, 0))],
        out_specs=pl.BlockSpec((BN, H), lambda i: (i, 0)),
        out_shape=jax.ShapeDtypeStruct((N_PAD, H), jnp.float32),
    )(h, w)


def _tc_update_body(p0_ref, p1_ref, h_ref, l2w, l2b, lw, lb, l1wn,
                    hn_ref, xln_ref):
    agg = p0_ref[...] + p1_ref[...]
    t = _ssp(jnp.dot(agg, l2w[...], preferred_element_type=jnp.float32)
             + l2b[...])
    y = jnp.dot(t, lw[...], preferred_element_type=jnp.float32) + lb[...]
    hn = h_ref[...] + y
    hn_ref[...] = hn
    xln_ref[...] = jnp.dot(hn, l1wn[...], preferred_element_type=jnp.float32)


def _tc_update(p0, p1, h, l2w, l2b, lw, lb, l1wn):
    wspec = lambda shp: pl.BlockSpec(shp, lambda i: (0, 0))
    return pl.pallas_call(
        _tc_update_body,
        grid=(N_PAD // BN,),
        in_specs=[pl.BlockSpec((BN, H), lambda i: (i, 0)),
                  pl.BlockSpec((BN, H), lambda i: (i, 0)),
                  pl.BlockSpec((BN, H), lambda i: (i, 0)),
                  wspec((H, H)), wspec((1, H)), wspec((H, H)), wspec((1, H)),
                  wspec((H, H))],
        out_specs=[pl.BlockSpec((BN, H), lambda i: (i, 0)),
                   pl.BlockSpec((BN, H), lambda i: (i, 0))],
        out_shape=[jax.ShapeDtypeStruct((N_PAD, H), jnp.float32),
                   jax.ShapeDtypeStruct((N_PAD, H), jnp.float32)],
    )(p0, p1, h, l2w, l2b, lw, lb, l1wn)


def _tc_readout_body(h2p_ref, fl1b, fl2w, fl2b, batch_ref, pw, pb,
                     out_ref, g_ref):
    t = _ssp(h2p_ref[...] + fl1b[...])
    h2 = jnp.dot(t, fl2w[...], preferred_element_type=jnp.float32) + fl2b[...]
    bt = batch_ref[...]
    oh = (bt == lax.broadcasted_iota(jnp.int32, (1, NGRAPH), 1))
    oh = oh.astype(jnp.float32)
    g_part = lax.dot_general(oh, h2, (((0,), (0,)), ((), ())),
                             preferred_element_type=jnp.float32)
    pid = pl.program_id(0)

    @pl.when(pid == 0)
    def _():
        g_ref[...] = g_part

    @pl.when(pid > 0)
    def _():
        g_ref[...] = g_ref[...] + g_part

    @pl.when(pid == pl.num_programs(0) - 1)
    def _():
        out_ref[...] = (jnp.dot(g_ref[...], pw[...],
                                preferred_element_type=jnp.float32) + pb[...])


def _tc_readout(h2p, fl1b, fl2w, fl2b, batch2d, pw, pb):
    wspec = lambda shp: pl.BlockSpec(shp, lambda i: (0, 0))
    return pl.pallas_call(
        _tc_readout_body,
        grid=(N_PAD // BN,),
        in_specs=[pl.BlockSpec((BN, H), lambda i: (i, 0)),
                  wspec((1, H)), wspec((H, H)), wspec((1, H)),
                  pl.BlockSpec((BN, 1), lambda i: (i, 0)),
                  wspec((H, 1)), wspec((1, 1))],
        out_specs=pl.BlockSpec((NGRAPH, 1), lambda i: (0, 0)),
        out_shape=jax.ShapeDtypeStruct((NGRAPH, 1), jnp.float32),
        scratch_shapes=[pltpu.VMEM((NGRAPH, H), jnp.float32)],
    )(h2p, fl1b, fl2w, fl2b, batch2d, pw, pb)


def kernel(z, pos, batch, edge_index, emb, mw1_0, mb1_0, mw2_0, mb2_0, l1w_0,
           l2w_0, l2b_0, lw_0, lb_0, mw1_1, mb1_1, mw2_1, mb2_1, l1w_1,
           l2w_1, l2b_1, lw_1, lb_1, fl1w, fl1b, fl2w, fl2b, pw, pb):
    src = edge_index[0].astype(jnp.int32)
    dst = edge_index[1].astype(jnp.int32)
    epad = E_PAD - E
    src_p = jnp.concatenate([src, jnp.zeros((epad,), jnp.int32)])
    dst_p = jnp.concatenate([dst, jnp.zeros((epad,), jnp.int32)])
    z_p = jnp.concatenate([z.astype(jnp.int32),
                           jnp.zeros((N_PAD - N,), jnp.int32)])
    batch_p = jnp.concatenate([batch.astype(jnp.int32),
                               jnp.full((N_PAD - N,), NGRAPH, jnp.int32)])
    zeros_tbl = jnp.zeros((N_PAD, H), jnp.float32)

    mw1p0 = jnp.zeros((16, H), jnp.float32).at[:NGAUSS].set(mw1_0)
    mw1p1 = jnp.zeros((16, H), jnp.float32).at[:NGAUSS].set(mw1_1)
    fl1wp = jnp.zeros((H, H), jnp.float32).at[:, :H // 2].set(fl1w)
    fl1bp = jnp.zeros((1, H), jnp.float32).at[0, :H // 2].set(fl1b)
    fl2wp = jnp.zeros((H, H), jnp.float32).at[:H // 2].set(fl2w)
    r2 = lambda v: v.reshape(1, -1)

    dist2, h0 = _sc_prep(pos[:, 0], pos[:, 1], pos[:, 2], src_p, dst_p,
                         z_p, emb)
    wf0, wf1 = _tc_wf(dist2.reshape(E_PAD, 1), mw1p0, r2(mb1_0), mw2_0,
                      r2(mb2_0), mw1p1, r2(mb1_1), mw2_1, r2(mb2_1))

    xl0 = _tc_xl(h0, l1w_0)
    agg0 = _sc_msgpass(xl0, wf0, src_p, dst_p, zeros_tbl)
    h1, xl1 = _tc_update(agg0[0], agg0[1], h0, l2w_0, r2(l2b_0), lw_0,
                         r2(lb_0), l1w_1)

    agg1 = _sc_msgpass(xl1, wf1, src_p, dst_p, zeros_tbl)
    h2, h2p = _tc_update(agg1[0], agg1[1], h1, l2w_1, r2(l2b_1), lw_1,
                         r2(lb_1), fl1wp)
    del h2

    out = _tc_readout(h2p, fl1bp, fl2wp, r2(fl2b), batch_p.reshape(N_PAD, 1),
                      pw, r2(pb))
    return out


# speedup vs baseline: 2.2324x; 1.1614x over previous
import functools

import jax
import jax.numpy as jnp
import numpy as np
from jax import lax
from jax.experimental import pallas as pl
from jax.experimental.pallas import tpu as pltpu
from jax.experimental.pallas import tpu_sc as plsc

N = 10000
E = 320000
H = 128
NGAUSS = 10
NGRAPH = 64
CUTOFF = 10.0

NW = 32
EPT = 10240
E_PAD = EPT * NW
CH = 128
NPT = 320
N_PAD = NPT * NW
BE = 1024
BN = 1024

_LOG2 = float(np.log(2.0))
_DELTA = CUTOFF / (NGAUSS - 1)
_COEFF = -0.5 / (_DELTA * _DELTA)
_OFFS = np.full((1, 16), 1e4, np.float32)
_OFFS[0, :NGAUSS] = np.linspace(0.0, CUTOFF, NGAUSS, dtype=np.float32)


def _ssp(x):
    return jnp.maximum(x, 0.0) + jnp.log1p(jnp.exp(-jnp.abs(x))) - _LOG2


def _sc_prep_body(px_hbm, py_hbm, pz_hbm, src_hbm, dst_hbm, z_hbm, emb_hbm,
                  d2_out, h0_out,
                  src_v, dst_v, sx_v, sy_v, sz_v, tx_v, ty_v, tz_v,
                  ux_v, uy_v, uz_v, vx_v, vy_v, vz_v,
                  d2_v, z_v, emb_v, sem, sem2):
    cid = lax.axis_index("c")
    sid = lax.axis_index("s")
    wid = cid * 16 + sid
    ebase = wid * EPT
    pltpu.sync_copy(src_hbm.at[pl.ds(ebase, EPT)], src_v)
    pltpu.sync_copy(dst_hbm.at[pl.ds(ebase, EPT)], dst_v)

    def fire(c, bufs, sem_c):
        isrc = src_v.at[pl.ds(c * CH, CH)]
        idst = dst_v.at[pl.ds(c * CH, CH)]
        return [pltpu.async_copy(px_hbm.at[isrc], bufs[0], sem_c),
                pltpu.async_copy(py_hbm.at[isrc], bufs[1], sem_c),
                pltpu.async_copy(pz_hbm.at[isrc], bufs[2], sem_c),
                pltpu.async_copy(px_hbm.at[idst], bufs[3], sem_c),
                pltpu.async_copy(py_hbm.at[idst], bufs[4], sem_c),
                pltpu.async_copy(pz_hbm.at[idst], bufs[5], sem_c)]

    bufs_a = (sx_v, sy_v, sz_v, tx_v, ty_v, tz_v)
    bufs_b = (ux_v, uy_v, uz_v, vx_v, vy_v, vz_v)

    def pair(k, carry):
        cps_a = fire(2 * k, bufs_a, sem)
        cps_b = fire(2 * k + 1, bufs_b, sem2)
        for cp in cps_a:
            cp.wait()
        for c, bufs in ((2 * k, bufs_a), (2 * k + 1, bufs_b)):
            if bufs is bufs_b:
                for cp in cps_b:
                    cp.wait()
            for v in range(CH // 16):
                sl = pl.ds(v * 16, 16)
                ddx = bufs[0][sl] - bufs[3][sl]
                ddy = bufs[1][sl] - bufs[4][sl]
                ddz = bufs[2][sl] - bufs[5][sl]
                d2_v[pl.ds(c * CH + v * 16, 16)] = (
                    ddx * ddx + ddy * ddy + ddz * ddz)
        return carry

    lax.fori_loop(0, EPT // CH // 2, pair, 0)
    pltpu.sync_copy(d2_v, d2_out.at[pl.ds(ebase, EPT)])

    nbase = wid * NPT
    pltpu.sync_copy(z_hbm.at[pl.ds(nbase, NPT)], z_v)
    for c0 in range(0, NPT, 80):
        pltpu.async_copy(emb_hbm.at[z_v.at[pl.ds(c0, 80)]],
                         emb_v.at[pl.ds(c0, 80)], sem).wait()
    pltpu.sync_copy(emb_v, h0_out.at[pl.ds(nbase, NPT)])


_sc_prep = functools.partial(
    pl.kernel,
    out_type=[jax.ShapeDtypeStruct((E_PAD,), jnp.float32),
              jax.ShapeDtypeStruct((N_PAD, H), jnp.float32)],
    mesh=plsc.VectorSubcoreMesh(core_axis_name="c", subcore_axis_name="s"),
    scratch_types=(
        [pltpu.VMEM((EPT,), jnp.int32),
         pltpu.VMEM((EPT,), jnp.int32)]
        + [pltpu.VMEM((CH,), jnp.float32) for _ in range(12)]
        + [pltpu.VMEM((EPT,), jnp.float32),
           pltpu.VMEM((NPT,), jnp.int32),
           pltpu.VMEM((NPT, H), jnp.float32),
           pltpu.SemaphoreType.DMA,
           pltpu.SemaphoreType.DMA]
    ),
)(_sc_prep_body)


CHM = 64


def _sc_msgpass_body(xl_hbm, wf_hbm, src_hbm, dst_hbm, zeros_hbm,
                     agg_out,
                     src_v, dc0, dc1, xr0, xr1, wf0, wf1, acc_sh,
                     sg0, sg1, sw0, sw1, ss0, ss1):
    cid = lax.axis_index("c")
    sid = lax.axis_index("s")
    wid = cid * 16 + sid
    rpt = N_PAD // 16
    r0 = sid * rpt
    pltpu.sync_copy(zeros_hbm.at[pl.ds(r0, rpt)], acc_sh.at[pl.ds(r0, rpt)])
    ebase = wid * EPT
    nch = EPT // CHM
    pltpu.sync_copy(src_hbm.at[pl.ds(ebase, EPT)], src_v)
    plsc.subcore_barrier()

    def mul(xr, wf):
        def row(r, c2):
            for v in range(8):
                sl = pl.ds(v * 16, 16)
                xr[r, sl] = xr[r, sl] * wf[r, sl]
            return c2

        lax.fori_loop(0, CHM, row, 0)

    def pair(k, carry):
        ch0 = 2 * k
        ch1 = 2 * k + 1
        g0 = pltpu.async_copy(xl_hbm.at[src_v.at[pl.ds(ch0 * CHM, CHM)]],
                              xr0, sg0)
        w0 = pltpu.async_copy(wf_hbm.at[pl.ds(ebase + ch0 * CHM, CHM)],
                              wf0, sw0)
        g1 = pltpu.async_copy(xl_hbm.at[src_v.at[pl.ds(ch1 * CHM, CHM)]],
                              xr1, sg1)
        w1 = pltpu.async_copy(wf_hbm.at[pl.ds(ch1 * CHM + ebase, CHM)],
                              wf1, sw1)
        pltpu.sync_copy(dst_hbm.at[pl.ds(ebase + ch0 * CHM, CHM)], dc0)
        pltpu.sync_copy(dst_hbm.at[pl.ds(ebase + ch1 * CHM, CHM)], dc1)
        g0.wait()
        w0.wait()
        mul(xr0, wf0)
        s0 = pltpu.async_copy(xr0, acc_sh.at[dc0], ss0, add=True)
        g1.wait()
        w1.wait()
        mul(xr1, wf1)
        s1 = pltpu.async_copy(xr1, acc_sh.at[dc1], ss1, add=True)
        s0.wait()
        s1.wait()
        return carry

    lax.fori_loop(0, nch // 2, pair, 0)
    plsc.subcore_barrier()
    pltpu.sync_copy(acc_sh.at[pl.ds(r0, rpt)],
                    agg_out.at[cid].at[pl.ds(r0, rpt)])


_sc_msgpass = functools.partial(
    pl.kernel,
    out_type=jax.ShapeDtypeStruct((2, N_PAD, H), jnp.float32),
    mesh=plsc.VectorSubcoreMesh(core_axis_name="c", subcore_axis_name="s"),
    scratch_types=[
        pltpu.VMEM((EPT,), jnp.int32),
        pltpu.VMEM((CHM,), jnp.int32),
        pltpu.VMEM((CHM,), jnp.int32),
        pltpu.VMEM((CHM, H), jnp.float32),
        pltpu.VMEM((CHM, H), jnp.float32),
        pltpu.VMEM((CHM, H), jnp.float32),
        pltpu.VMEM((CHM, H), jnp.float32),
        pltpu.VMEM_SHARED((N_PAD, H), jnp.float32),
        pltpu.SemaphoreType.DMA,
        pltpu.SemaphoreType.DMA,
        pltpu.SemaphoreType.DMA,
        pltpu.SemaphoreType.DMA,
        pltpu.SemaphoreType.DMA,
        pltpu.SemaphoreType.DMA,
    ],
)(_sc_msgpass_body)


EROWS = E_PAD // H
BP = 256


def _tc_cdist_body(d2_ref, dist_ref, c_ref):
    d2 = d2_ref[...]
    dist = jnp.sqrt(d2 + 1e-12)
    c = 0.5 * (jnp.cos(dist * (np.pi / CUTOFF)) + 1.0)
    rows = (pl.program_id(0) * BP
            + lax.broadcasted_iota(jnp.int32, (BP, 1), 0))
    c = jnp.where(rows < E // H, c, 0.0)
    dist_ref[...] = dist
    c_ref[...] = c


def _tc_cdist(d2pk):
    spec = pl.BlockSpec((BP, H), lambda i: (i, 0))
    return pl.pallas_call(
        _tc_cdist_body,
        grid=(EROWS // BP,),
        in_specs=[spec],
        out_specs=[spec, spec],
        out_shape=[jax.ShapeDtypeStruct((EROWS, H), jnp.float32),
                   jax.ShapeDtypeStruct((EROWS, H), jnp.float32)],
    )(d2pk)


def _tc_wf_body(dist_ref, c_ref, w1, b1, w2, b2, wf_ref):
    dist = dist_ref[...]
    lane = lax.broadcasted_iota(jnp.int32, (1, 16), 1)
    offs = jnp.where(lane < NGAUSS, lane.astype(jnp.float32) * _DELTA, 1e4)
    diff = dist - offs
    rbf = jnp.exp(_COEFF * (diff * diff))
    t = _ssp(jnp.dot(rbf, w1[...], preferred_element_type=jnp.float32)
             + b1[...])
    wf_ref[...] = (jnp.dot(t, w2[...], preferred_element_type=jnp.float32)
                   + b2[...]) * c_ref[...]


def _tc_wf(dist_col, c_col, w1, b1, w2, b2):
    g = E_PAD // BE
    wspec = lambda shp: pl.BlockSpec(shp, lambda i: (0, 0))
    return pl.pallas_call(
        _tc_wf_body,
        grid=(g,),
        in_specs=[
            pl.BlockSpec((BE, 1), lambda i: (i, 0)),
            pl.BlockSpec((BE, 1), lambda i: (i, 0)),
            wspec((16, H)), wspec((1, H)), wspec((H, H)), wspec((1, H)),
        ],
        out_specs=pl.BlockSpec((BE, H), lambda i: (i, 0)),
        out_shape=jax.ShapeDtypeStruct((E_PAD, H), jnp.float32),
    )(dist_col, c_col, w1, b1, w2, b2)


def _tc_xl_body(h_ref, w_ref, o_ref):
    o_ref[...] = jnp.dot(h_ref[...], w_ref[...],
                         preferred_element_type=jnp.float32)


def _tc_xl(h, w):
    return pl.pallas_call(
        _tc_xl_body,
        grid=(N_PAD // BN,),
        in_specs=[pl.BlockSpec((BN, H), lambda i: (i, 0)),
                  pl.BlockSpec((H, H), lambda i: (0, 0))],
        out_specs=pl.BlockSpec((BN, H), lambda i: (i, 0)),
        out_shape=jax.ShapeDtypeStruct((N_PAD, H), jnp.float32),
    )(h, w)


def _tc_update_body(p0_ref, p1_ref, h_ref, l2w, l2b, lw, lb, l1wn,
                    hn_ref, xln_ref):
    agg = p0_ref[...] + p1_ref[...]
    t = _ssp(jnp.dot(agg, l2w[...], preferred_element_type=jnp.float32)
             + l2b[...])
    y = jnp.dot(t, lw[...], preferred_element_type=jnp.float32) + lb[...]
    hn = h_ref[...] + y
    hn_ref[...] = hn
    xln_ref[...] = jnp.dot(hn, l1wn[...], preferred_element_type=jnp.float32)


def _tc_update(p0, p1, h, l2w, l2b, lw, lb, l1wn):
    wspec = lambda shp: pl.BlockSpec(shp, lambda i: (0, 0))
    return pl.pallas_call(
        _tc_update_body,
        grid=(N_PAD // BN,),
        in_specs=[pl.BlockSpec((BN, H), lambda i: (i, 0)),
                  pl.BlockSpec((BN, H), lambda i: (i, 0)),
                  pl.BlockSpec((BN, H), lambda i: (i, 0)),
                  wspec((H, H)), wspec((1, H)), wspec((H, H)), wspec((1, H)),
                  wspec((H, H))],
        out_specs=[pl.BlockSpec((BN, H), lambda i: (i, 0)),
                   pl.BlockSpec((BN, H), lambda i: (i, 0))],
        out_shape=[jax.ShapeDtypeStruct((N_PAD, H), jnp.float32),
                   jax.ShapeDtypeStruct((N_PAD, H), jnp.float32)],
    )(p0, p1, h, l2w, l2b, lw, lb, l1wn)


def _tc_readout_body(h2p_ref, fl1b, fl2w, fl2b, batch_ref, pw, pb,
                     out_ref, g_ref):
    t = _ssp(h2p_ref[...] + fl1b[...])
    h2 = jnp.dot(t, fl2w[...], preferred_element_type=jnp.float32) + fl2b[...]
    bt = batch_ref[...]
    oh = (bt == lax.broadcasted_iota(jnp.int32, (1, NGRAPH), 1))
    oh = oh.astype(jnp.float32)
    g_part = lax.dot_general(oh, h2, (((0,), (0,)), ((), ())),
                             preferred_element_type=jnp.float32)
    pid = pl.program_id(0)

    @pl.when(pid == 0)
    def _():
        g_ref[...] = g_part

    @pl.when(pid > 0)
    def _():
        g_ref[...] = g_ref[...] + g_part

    @pl.when(pid == pl.num_programs(0) - 1)
    def _():
        out_ref[...] = (jnp.dot(g_ref[...], pw[...],
                                preferred_element_type=jnp.float32) + pb[...])


def _tc_readout(h2p, fl1b, fl2w, fl2b, batch2d, pw, pb):
    wspec = lambda shp: pl.BlockSpec(shp, lambda i: (0, 0))
    return pl.pallas_call(
        _tc_readout_body,
        grid=(N_PAD // BN,),
        in_specs=[pl.BlockSpec((BN, H), lambda i: (i, 0)),
                  wspec((1, H)), wspec((H, H)), wspec((1, H)),
                  pl.BlockSpec((BN, 1), lambda i: (i, 0)),
                  wspec((H, 1)), wspec((1, 1))],
        out_specs=pl.BlockSpec((NGRAPH, 1), lambda i: (0, 0)),
        out_shape=jax.ShapeDtypeStruct((NGRAPH, 1), jnp.float32),
        scratch_shapes=[pltpu.VMEM((NGRAPH, H), jnp.float32)],
    )(h2p, fl1b, fl2w, fl2b, batch2d, pw, pb)


def kernel(z, pos, batch, edge_index, emb, mw1_0, mb1_0, mw2_0, mb2_0, l1w_0,
           l2w_0, l2b_0, lw_0, lb_0, mw1_1, mb1_1, mw2_1, mb2_1, l1w_1,
           l2w_1, l2b_1, lw_1, lb_1, fl1w, fl1b, fl2w, fl2b, pw, pb):
    src = edge_index[0].astype(jnp.int32)
    dst = edge_index[1].astype(jnp.int32)
    epad = E_PAD - E
    src_p = jnp.concatenate([src, jnp.zeros((epad,), jnp.int32)])
    dst_p = jnp.concatenate([dst, jnp.zeros((epad,), jnp.int32)])
    z_p = jnp.concatenate([z.astype(jnp.int32),
                           jnp.zeros((N_PAD - N,), jnp.int32)])
    batch_p = jnp.concatenate([batch.astype(jnp.int32),
                               jnp.full((N_PAD - N,), NGRAPH, jnp.int32)])
    zeros_tbl = jnp.zeros((N_PAD, H), jnp.float32)

    mw1p0 = jnp.zeros((16, H), jnp.float32).at[:NGAUSS].set(mw1_0)
    mw1p1 = jnp.zeros((16, H), jnp.float32).at[:NGAUSS].set(mw1_1)
    fl1wp = jnp.zeros((H, H), jnp.float32).at[:, :H // 2].set(fl1w)
    fl1bp = jnp.zeros((1, H), jnp.float32).at[0, :H // 2].set(fl1b)
    fl2wp = jnp.zeros((H, H), jnp.float32).at[:H // 2].set(fl2w)
    r2 = lambda v: v.reshape(1, -1)

    dist2, h0 = _sc_prep(pos[:, 0], pos[:, 1], pos[:, 2], src_p, dst_p,
                         z_p, emb)
    dist_pk, c_pk = _tc_cdist(dist2.reshape(EROWS, H))
    dist_col = dist_pk.reshape(E_PAD, 1)
    c_col = c_pk.reshape(E_PAD, 1)
    wf0 = _tc_wf(dist_col, c_col, mw1p0, r2(mb1_0), mw2_0, r2(mb2_0))

    xl0 = _tc_xl(h0, l1w_0)
    agg0 = _sc_msgpass(xl0, wf0, src_p, dst_p, zeros_tbl)
    wf1 = _tc_wf(dist_col, c_col, mw1p1, r2(mb1_1), mw2_1, r2(mb2_1))
    h1, xl1 = _tc_update(agg0[0], agg0[1], h0, l2w_0, r2(l2b_0), lw_0,
                         r2(lb_0), l1w_1)

    agg1 = _sc_msgpass(xl1, wf1, src_p, dst_p, zeros_tbl)
    h2, h2p = _tc_update(agg1[0], agg1[1], h1, l2w_1, r2(l2b_1), lw_1,
                         r2(lb_1), fl1wp)
    del h2

    out = _tc_readout(h2p, fl1bp, fl2wp, r2(fl2b), batch_p.reshape(N_PAD, 1),
                      pw, r2(pb))
    return out


# speedup vs baseline: 2.4197x; 1.0839x over previous
import functools

import jax
import jax.numpy as jnp
import numpy as np
from jax import lax
from jax.experimental import pallas as pl
from jax.experimental.pallas import tpu as pltpu
from jax.experimental.pallas import tpu_sc as plsc

N = 10000
E = 320000
H = 128
NGAUSS = 10
NGRAPH = 64
CUTOFF = 10.0

NW = 32
EPT = 10240
E_PAD = EPT * NW
CH = 128
NPT = 320
N_PAD = NPT * NW
BE = 1024
BN = 1024

_LOG2 = float(np.log(2.0))
_DELTA = CUTOFF / (NGAUSS - 1)
_COEFF = -0.5 / (_DELTA * _DELTA)
_OFFS = np.full((1, 16), 1e4, np.float32)
_OFFS[0, :NGAUSS] = np.linspace(0.0, CUTOFF, NGAUSS, dtype=np.float32)


def _ssp(x):
    return jnp.maximum(x, 0.0) + jnp.log1p(jnp.exp(-jnp.abs(x))) - _LOG2


def _sc_prep_body(px_hbm, py_hbm, pz_hbm, src_hbm, dst_hbm, z_hbm, emb_hbm,
                  d2_out, h0_out,
                  src_v, dst_v, sx_v, sy_v, sz_v, tx_v, ty_v, tz_v,
                  ux_v, uy_v, uz_v, vx_v, vy_v, vz_v,
                  d2_v, z_v, emb_v, sem, sem2):
    cid = lax.axis_index("c")
    sid = lax.axis_index("s")
    wid = cid * 16 + sid
    ebase = wid * EPT
    pltpu.sync_copy(src_hbm.at[pl.ds(ebase, EPT)], src_v)
    pltpu.sync_copy(dst_hbm.at[pl.ds(ebase, EPT)], dst_v)

    def fire(c, bufs, sem_c):
        isrc = src_v.at[pl.ds(c * CH, CH)]
        idst = dst_v.at[pl.ds(c * CH, CH)]
        return [pltpu.async_copy(px_hbm.at[isrc], bufs[0], sem_c),
                pltpu.async_copy(py_hbm.at[isrc], bufs[1], sem_c),
                pltpu.async_copy(pz_hbm.at[isrc], bufs[2], sem_c),
                pltpu.async_copy(px_hbm.at[idst], bufs[3], sem_c),
                pltpu.async_copy(py_hbm.at[idst], bufs[4], sem_c),
                pltpu.async_copy(pz_hbm.at[idst], bufs[5], sem_c)]

    bufs_a = (sx_v, sy_v, sz_v, tx_v, ty_v, tz_v)
    bufs_b = (ux_v, uy_v, uz_v, vx_v, vy_v, vz_v)

    def pair(k, carry):
        cps_a = fire(2 * k, bufs_a, sem)
        cps_b = fire(2 * k + 1, bufs_b, sem2)
        for cp in cps_a:
            cp.wait()
        for c, bufs in ((2 * k, bufs_a), (2 * k + 1, bufs_b)):
            if bufs is bufs_b:
                for cp in cps_b:
                    cp.wait()
            for v in range(CH // 16):
                sl = pl.ds(v * 16, 16)
                ddx = bufs[0][sl] - bufs[3][sl]
                ddy = bufs[1][sl] - bufs[4][sl]
                ddz = bufs[2][sl] - bufs[5][sl]
                d2_v[pl.ds(c * CH + v * 16, 16)] = (
                    ddx * ddx + ddy * ddy + ddz * ddz)
        return carry

    lax.fori_loop(0, EPT // CH // 2, pair, 0)
    pltpu.sync_copy(d2_v, d2_out.at[pl.ds(ebase, EPT)])

    nbase = wid * NPT
    pltpu.sync_copy(z_hbm.at[pl.ds(nbase, NPT)], z_v)
    for c0 in range(0, NPT, 80):
        pltpu.async_copy(emb_hbm.at[z_v.at[pl.ds(c0, 80)]],
                         emb_v.at[pl.ds(c0, 80)], sem).wait()
    pltpu.sync_copy(emb_v, h0_out.at[pl.ds(nbase, NPT)])


_sc_prep = functools.partial(
    pl.kernel,
    out_type=[jax.ShapeDtypeStruct((E_PAD,), jnp.float32),
              jax.ShapeDtypeStruct((N_PAD, H), jnp.float32)],
    mesh=plsc.VectorSubcoreMesh(core_axis_name="c", subcore_axis_name="s"),
    scratch_types=(
        [pltpu.VMEM((EPT,), jnp.int32),
         pltpu.VMEM((EPT,), jnp.int32)]
        + [pltpu.VMEM((CH,), jnp.float32) for _ in range(12)]
        + [pltpu.VMEM((EPT,), jnp.float32),
           pltpu.VMEM((NPT,), jnp.int32),
           pltpu.VMEM((NPT, H), jnp.float32),
           pltpu.SemaphoreType.DMA,
           pltpu.SemaphoreType.DMA]
    ),
)(_sc_prep_body)


CHM = 64
E_C0 = 227328
E_C1 = E_PAD - E_C0
T0 = E_C0 // 16
T1 = E_C1 // 16


def _sc_msgpass_body(xl_hbm, wf_hbm, src_hbm, dst_hbm, zeros_hbm,
                     agg_out,
                     src_v, dc0, dc1, xr0, xr1, wf0, wf1, acc_sh,
                     sg0, sg1, sw0, sw1, ss0, ss1):
    cid = lax.axis_index("c")
    sid = lax.axis_index("s")
    rpt = N_PAD // 16
    r0 = sid * rpt
    pltpu.sync_copy(zeros_hbm.at[pl.ds(r0, rpt)], acc_sh.at[pl.ds(r0, rpt)])

    def mul(xr, wf):
        def row(r, c2):
            for v in range(8):
                sl = pl.ds(v * 16, 16)
                xr[r, sl] = xr[r, sl] * wf[r, sl]
            return c2

        lax.fori_loop(0, CHM, row, 0)

    def run(ebase, npairs):
        def pair(k, carry):
            ch0 = 2 * k
            ch1 = 2 * k + 1
            g0 = pltpu.async_copy(xl_hbm.at[src_v.at[pl.ds(ch0 * CHM, CHM)]],
                                  xr0, sg0)
            w0 = pltpu.async_copy(wf_hbm.at[pl.ds(ebase + ch0 * CHM, CHM)],
                                  wf0, sw0)
            g1 = pltpu.async_copy(xl_hbm.at[src_v.at[pl.ds(ch1 * CHM, CHM)]],
                                  xr1, sg1)
            w1 = pltpu.async_copy(wf_hbm.at[pl.ds(ch1 * CHM + ebase, CHM)],
                                  wf1, sw1)
            pltpu.sync_copy(dst_hbm.at[pl.ds(ebase + ch0 * CHM, CHM)], dc0)
            pltpu.sync_copy(dst_hbm.at[pl.ds(ebase + ch1 * CHM, CHM)], dc1)
            g0.wait()
            w0.wait()
            mul(xr0, wf0)
            s0 = pltpu.async_copy(xr0, acc_sh.at[dc0], ss0, add=True)
            g1.wait()
            w1.wait()
            mul(xr1, wf1)
            s1 = pltpu.async_copy(xr1, acc_sh.at[dc1], ss1, add=True)
            s0.wait()
            s1.wait()
            return carry

        lax.fori_loop(0, npairs, pair, 0)

    @pl.when(cid == 0)
    def _():
        eb = sid * T0
        pltpu.sync_copy(src_hbm.at[pl.ds(eb, T0)], src_v.at[pl.ds(0, T0)])
        plsc.subcore_barrier()
        run(eb, T0 // (2 * CHM))

    @pl.when(cid == 1)
    def _():
        eb = E_C0 + sid * T1
        pltpu.sync_copy(src_hbm.at[pl.ds(eb, T1)], src_v.at[pl.ds(0, T1)])
        plsc.subcore_barrier()
        run(eb, T1 // (2 * CHM))

    plsc.subcore_barrier()
    pltpu.sync_copy(acc_sh.at[pl.ds(r0, rpt)],
                    agg_out.at[cid].at[pl.ds(r0, rpt)])


_sc_msgpass = functools.partial(
    pl.kernel,
    out_type=jax.ShapeDtypeStruct((2, N_PAD, H), jnp.float32),
    mesh=plsc.VectorSubcoreMesh(core_axis_name="c", subcore_axis_name="s"),
    scratch_types=[
        pltpu.VMEM((T0,), jnp.int32),
        pltpu.VMEM((CHM,), jnp.int32),
        pltpu.VMEM((CHM,), jnp.int32),
        pltpu.VMEM((CHM, H), jnp.float32),
        pltpu.VMEM((CHM, H), jnp.float32),
        pltpu.VMEM((CHM, H), jnp.float32),
        pltpu.VMEM((CHM, H), jnp.float32),
        pltpu.VMEM_SHARED((N_PAD, H), jnp.float32),
        pltpu.SemaphoreType.DMA,
        pltpu.SemaphoreType.DMA,
        pltpu.SemaphoreType.DMA,
        pltpu.SemaphoreType.DMA,
        pltpu.SemaphoreType.DMA,
        pltpu.SemaphoreType.DMA,
    ],
)(_sc_msgpass_body)


EROWS = E_PAD // H
BP = 256


def _tc_cdist_body(d2_ref, dist_ref, c_ref):
    d2 = d2_ref[...]
    dist = jnp.sqrt(d2 + 1e-12)
    c = 0.5 * (jnp.cos(dist * (np.pi / CUTOFF)) + 1.0)
    rows = (pl.program_id(0) * BP
            + lax.broadcasted_iota(jnp.int32, (BP, 1), 0))
    c = jnp.where(rows < E // H, c, 0.0)
    dist_ref[...] = dist
    c_ref[...] = c


def _tc_cdist(d2pk):
    spec = pl.BlockSpec((BP, H), lambda i: (i, 0))
    return pl.pallas_call(
        _tc_cdist_body,
        grid=(EROWS // BP,),
        in_specs=[spec],
        out_specs=[spec, spec],
        out_shape=[jax.ShapeDtypeStruct((EROWS, H), jnp.float32),
                   jax.ShapeDtypeStruct((EROWS, H), jnp.float32)],
    )(d2pk)


def _tc_wf_body(dist_ref, c_ref, w1, b1, w2, b2, wf_ref):
    dist = dist_ref[...]
    lane = lax.broadcasted_iota(jnp.int32, (1, 16), 1)
    offs = jnp.where(lane < NGAUSS, lane.astype(jnp.float32) * _DELTA, 1e4)
    diff = dist - offs
    rbf = jnp.exp(_COEFF * (diff * diff))
    t = _ssp(jnp.dot(rbf, w1[...], preferred_element_type=jnp.float32)
             + b1[...])
    wf_ref[...] = (jnp.dot(t, w2[...], preferred_element_type=jnp.float32)
                   + b2[...]) * c_ref[...]


def _tc_wf(dist_col, c_col, w1, b1, w2, b2):
    g = E_PAD // BE
    wspec = lambda shp: pl.BlockSpec(shp, lambda i: (0, 0))
    return pl.pallas_call(
        _tc_wf_body,
        grid=(g,),
        in_specs=[
            pl.BlockSpec((BE, 1), lambda i: (i, 0)),
            pl.BlockSpec((BE, 1), lambda i: (i, 0)),
            wspec((16, H)), wspec((1, H)), wspec((H, H)), wspec((1, H)),
        ],
        out_specs=pl.BlockSpec((BE, H), lambda i: (i, 0)),
        out_shape=jax.ShapeDtypeStruct((E_PAD, H), jnp.float32),
    )(dist_col, c_col, w1, b1, w2, b2)


def _tc_xl_body(h_ref, w_ref, o_ref):
    o_ref[...] = jnp.dot(h_ref[...], w_ref[...],
                         preferred_element_type=jnp.float32)


def _tc_xl(h, w):
    return pl.pallas_call(
        _tc_xl_body,
        grid=(N_PAD // BN,),
        in_specs=[pl.BlockSpec((BN, H), lambda i: (i, 0)),
                  pl.BlockSpec((H, H), lambda i: (0, 0))],
        out_specs=pl.BlockSpec((BN, H), lambda i: (i, 0)),
        out_shape=jax.ShapeDtypeStruct((N_PAD, H), jnp.float32),
    )(h, w)


def _tc_update_body(p0_ref, p1_ref, h_ref, l2w, l2b, lw, lb, l1wn,
                    hn_ref, xln_ref):
    agg = p0_ref[...] + p1_ref[...]
    t = _ssp(jnp.dot(agg, l2w[...], preferred_element_type=jnp.float32)
             + l2b[...])
    y = jnp.dot(t, lw[...], preferred_element_type=jnp.float32) + lb[...]
    hn = h_ref[...] + y
    hn_ref[...] = hn
    xln_ref[...] = jnp.dot(hn, l1wn[...], preferred_element_type=jnp.float32)


def _tc_update(p0, p1, h, l2w, l2b, lw, lb, l1wn):
    wspec = lambda shp: pl.BlockSpec(shp, lambda i: (0, 0))
    return pl.pallas_call(
        _tc_update_body,
        grid=(N_PAD // BN,),
        in_specs=[pl.BlockSpec((BN, H), lambda i: (i, 0)),
                  pl.BlockSpec((BN, H), lambda i: (i, 0)),
                  pl.BlockSpec((BN, H), lambda i: (i, 0)),
                  wspec((H, H)), wspec((1, H)), wspec((H, H)), wspec((1, H)),
                  wspec((H, H))],
        out_specs=[pl.BlockSpec((BN, H), lambda i: (i, 0)),
                   pl.BlockSpec((BN, H), lambda i: (i, 0))],
        out_shape=[jax.ShapeDtypeStruct((N_PAD, H), jnp.float32),
                   jax.ShapeDtypeStruct((N_PAD, H), jnp.float32)],
    )(p0, p1, h, l2w, l2b, lw, lb, l1wn)


def _tc_readout_body(h2p_ref, fl1b, fl2w, fl2b, batch_ref, pw, pb,
                     out_ref, g_ref):
    t = _ssp(h2p_ref[...] + fl1b[...])
    h2 = jnp.dot(t, fl2w[...], preferred_element_type=jnp.float32) + fl2b[...]
    bt = batch_ref[...]
    oh = (bt == lax.broadcasted_iota(jnp.int32, (1, NGRAPH), 1))
    oh = oh.astype(jnp.float32)
    g_part = lax.dot_general(oh, h2, (((0,), (0,)), ((), ())),
                             preferred_element_type=jnp.float32)
    pid = pl.program_id(0)

    @pl.when(pid == 0)
    def _():
        g_ref[...] = g_part

    @pl.when(pid > 0)
    def _():
        g_ref[...] = g_ref[...] + g_part

    @pl.when(pid == pl.num_programs(0) - 1)
    def _():
        out_ref[...] = (jnp.dot(g_ref[...], pw[...],
                                preferred_element_type=jnp.float32) + pb[...])


def _tc_readout(h2p, fl1b, fl2w, fl2b, batch2d, pw, pb):
    wspec = lambda shp: pl.BlockSpec(shp, lambda i: (0, 0))
    return pl.pallas_call(
        _tc_readout_body,
        grid=(N_PAD // BN,),
        in_specs=[pl.BlockSpec((BN, H), lambda i: (i, 0)),
                  wspec((1, H)), wspec((H, H)), wspec((1, H)),
                  pl.BlockSpec((BN, 1), lambda i: (i, 0)),
                  wspec((H, 1)), wspec((1, 1))],
        out_specs=pl.BlockSpec((NGRAPH, 1), lambda i: (0, 0)),
        out_shape=jax.ShapeDtypeStruct((NGRAPH, 1), jnp.float32),
        scratch_shapes=[pltpu.VMEM((NGRAPH, H), jnp.float32)],
    )(h2p, fl1b, fl2w, fl2b, batch2d, pw, pb)


def kernel(z, pos, batch, edge_index, emb, mw1_0, mb1_0, mw2_0, mb2_0, l1w_0,
           l2w_0, l2b_0, lw_0, lb_0, mw1_1, mb1_1, mw2_1, mb2_1, l1w_1,
           l2w_1, l2b_1, lw_1, lb_1, fl1w, fl1b, fl2w, fl2b, pw, pb):
    src = edge_index[0].astype(jnp.int32)
    dst = edge_index[1].astype(jnp.int32)
    epad = E_PAD - E
    src_p = jnp.concatenate([src, jnp.zeros((epad,), jnp.int32)])
    dst_p = jnp.concatenate([dst, jnp.zeros((epad,), jnp.int32)])
    z_p = jnp.concatenate([z.astype(jnp.int32),
                           jnp.zeros((N_PAD - N,), jnp.int32)])
    batch_p = jnp.concatenate([batch.astype(jnp.int32),
                               jnp.full((N_PAD - N,), NGRAPH, jnp.int32)])
    zeros_tbl = jnp.zeros((N_PAD, H), jnp.float32)

    mw1p0 = jnp.zeros((16, H), jnp.float32).at[:NGAUSS].set(mw1_0)
    mw1p1 = jnp.zeros((16, H), jnp.float32).at[:NGAUSS].set(mw1_1)
    fl1wp = jnp.zeros((H, H), jnp.float32).at[:, :H // 2].set(fl1w)
    fl1bp = jnp.zeros((1, H), jnp.float32).at[0, :H // 2].set(fl1b)
    fl2wp = jnp.zeros((H, H), jnp.float32).at[:H // 2].set(fl2w)
    r2 = lambda v: v.reshape(1, -1)

    dist2, h0 = _sc_prep(pos[:, 0], pos[:, 1], pos[:, 2], src_p, dst_p,
                         z_p, emb)
    dist_pk, c_pk = _tc_cdist(dist2.reshape(EROWS, H))
    dist_col = dist_pk.reshape(E_PAD, 1)
    c_col = c_pk.reshape(E_PAD, 1)
    wf0 = _tc_wf(dist_col, c_col, mw1p0, r2(mb1_0), mw2_0, r2(mb2_0))

    xl0 = _tc_xl(h0, l1w_0)
    agg0 = _sc_msgpass(xl0, wf0, src_p, dst_p, zeros_tbl)
    wf1 = _tc_wf(dist_col, c_col, mw1p1, r2(mb1_1), mw2_1, r2(mb2_1))
    h1, xl1 = _tc_update(agg0[0], agg0[1], h0, l2w_0, r2(l2b_0), lw_0,
                         r2(lb_0), l1w_1)

    agg1 = _sc_msgpass(xl1, wf1, src_p, dst_p, zeros_tbl)
    h2, h2p = _tc_update(agg1[0], agg1[1], h1, l2w_1, r2(l2b_1), lw_1,
                         r2(lb_1), fl1wp)
    del h2

    out = _tc_readout(h2p, fl1bp, fl2wp, r2(fl2b), batch_p.reshape(N_PAD, 1),
                      pw, r2(pb))
    return out


# speedup vs baseline: 2.5195x; 1.0413x over previous
import functools

import jax
import jax.numpy as jnp
import numpy as np
from jax import lax
from jax.experimental import pallas as pl
from jax.experimental.pallas import tpu as pltpu
from jax.experimental.pallas import tpu_sc as plsc

N = 10000
E = 320000
H = 128
NGAUSS = 10
NGRAPH = 64
CUTOFF = 10.0

NW = 32
EPT = 10240
E_PAD = EPT * NW
CH = 128
NPT = 320
N_PAD = NPT * NW
BE = 1024
BN = 1024

_LOG2 = float(np.log(2.0))
_DELTA = CUTOFF / (NGAUSS - 1)
_COEFF = -0.5 / (_DELTA * _DELTA)
_OFFS = np.full((1, 16), 1e4, np.float32)
_OFFS[0, :NGAUSS] = np.linspace(0.0, CUTOFF, NGAUSS, dtype=np.float32)


def _ssp(x):
    return jnp.maximum(x, 0.0) + jnp.log1p(jnp.exp(-jnp.abs(x))) - _LOG2


PT0 = 12800
PT1 = 7680

def _sc_prep_body(px_hbm, py_hbm, pz_hbm, src_hbm, dst_hbm, z_hbm, emb_hbm,
                  d2_out, h0_out,
                  src_v, dst_v, sx_v, sy_v, sz_v, tx_v, ty_v, tz_v,
                  ux_v, uy_v, uz_v, vx_v, vy_v, vz_v,
                  d2_v, z_v, emb_v, sem, sem2):
    cid = lax.axis_index("c")
    sid = lax.axis_index("s")
    wid = cid * 16 + sid
    ebase = jnp.where(cid == 0, sid * PT0, PT0 * 16 + sid * PT1)
    ept_c = jnp.where(cid == 0, PT0, PT1)

    def fire(c, bufs, sem_c):
        isrc = src_v.at[pl.ds(c * CH, CH)]
        idst = dst_v.at[pl.ds(c * CH, CH)]
        return [pltpu.async_copy(px_hbm.at[isrc], bufs[0], sem_c),
                pltpu.async_copy(py_hbm.at[isrc], bufs[1], sem_c),
                pltpu.async_copy(pz_hbm.at[isrc], bufs[2], sem_c),
                pltpu.async_copy(px_hbm.at[idst], bufs[3], sem_c),
                pltpu.async_copy(py_hbm.at[idst], bufs[4], sem_c),
                pltpu.async_copy(pz_hbm.at[idst], bufs[5], sem_c)]

    bufs_a = (sx_v, sy_v, sz_v, tx_v, ty_v, tz_v)
    bufs_b = (ux_v, uy_v, uz_v, vx_v, vy_v, vz_v)

    def pair(k, carry):
        cps_a = fire(2 * k, bufs_a, sem)
        cps_b = fire(2 * k + 1, bufs_b, sem2)
        for cp in cps_a:
            cp.wait()
        for c, bufs in ((2 * k, bufs_a), (2 * k + 1, bufs_b)):
            if bufs is bufs_b:
                for cp in cps_b:
                    cp.wait()
            for v in range(CH // 16):
                sl = pl.ds(v * 16, 16)
                ddx = bufs[0][sl] - bufs[3][sl]
                ddy = bufs[1][sl] - bufs[4][sl]
                ddz = bufs[2][sl] - bufs[5][sl]
                d2_v[pl.ds(c * CH + v * 16, 16)] = (
                    ddx * ddx + ddy * ddy + ddz * ddz)
        return carry

    @pl.when(cid == 0)
    def _():
        pltpu.sync_copy(src_hbm.at[pl.ds(sid * PT0, PT0)],
                        src_v.at[pl.ds(0, PT0)])
        pltpu.sync_copy(dst_hbm.at[pl.ds(sid * PT0, PT0)],
                        dst_v.at[pl.ds(0, PT0)])
        lax.fori_loop(0, PT0 // CH // 2, pair, 0)
        pltpu.sync_copy(d2_v.at[pl.ds(0, PT0)],
                        d2_out.at[pl.ds(sid * PT0, PT0)])

    @pl.when(cid == 1)
    def _():
        eb = PT0 * 16 + sid * PT1
        pltpu.sync_copy(src_hbm.at[pl.ds(eb, PT1)], src_v.at[pl.ds(0, PT1)])
        pltpu.sync_copy(dst_hbm.at[pl.ds(eb, PT1)], dst_v.at[pl.ds(0, PT1)])
        lax.fori_loop(0, PT1 // CH // 2, pair, 0)
        pltpu.sync_copy(d2_v.at[pl.ds(0, PT1)], d2_out.at[pl.ds(eb, PT1)])

    nbase = wid * NPT
    pltpu.sync_copy(z_hbm.at[pl.ds(nbase, NPT)], z_v)
    for c0 in range(0, NPT, 80):
        pltpu.async_copy(emb_hbm.at[z_v.at[pl.ds(c0, 80)]],
                         emb_v.at[pl.ds(c0, 80)], sem).wait()
    pltpu.sync_copy(emb_v, h0_out.at[pl.ds(nbase, NPT)])


_sc_prep = functools.partial(
    pl.kernel,
    out_type=[jax.ShapeDtypeStruct((E_PAD,), jnp.float32),
              jax.ShapeDtypeStruct((N_PAD, H), jnp.float32)],
    mesh=plsc.VectorSubcoreMesh(core_axis_name="c", subcore_axis_name="s"),
    scratch_types=(
        [pltpu.VMEM((PT0,), jnp.int32),
         pltpu.VMEM((PT0,), jnp.int32)]
        + [pltpu.VMEM((CH,), jnp.float32) for _ in range(12)]
        + [pltpu.VMEM((PT0,), jnp.float32),
           pltpu.VMEM((NPT,), jnp.int32),
           pltpu.VMEM((NPT, H), jnp.float32),
           pltpu.SemaphoreType.DMA,
           pltpu.SemaphoreType.DMA]
    ),
)(_sc_prep_body)


CHM = 64
E_C0 = 245760
E_C1 = E_PAD - E_C0
T0 = E_C0 // 16
T1 = E_C1 // 16


def _sc_msgpass_body(xl_hbm, wf_hbm, src_hbm, dst_hbm, zeros_hbm,
                     agg_out,
                     src_v, dc0, dc1, xr0, xr1, wf0, wf1, acc_sh,
                     sg0, sg1, sw0, sw1, ss0, ss1):
    cid = lax.axis_index("c")
    sid = lax.axis_index("s")
    rpt = N_PAD // 16
    r0 = sid * rpt
    pltpu.sync_copy(zeros_hbm.at[pl.ds(r0, rpt)], acc_sh.at[pl.ds(r0, rpt)])

    def mul(xr, wf):
        def row(r, c2):
            for v in range(8):
                sl = pl.ds(v * 16, 16)
                xr[r, sl] = xr[r, sl] * wf[r, sl]
            return c2

        lax.fori_loop(0, CHM, row, 0)

    def run(ebase, npairs):
        def pair(k, carry):
            ch0 = 2 * k
            ch1 = 2 * k + 1
            g0 = pltpu.async_copy(xl_hbm.at[src_v.at[pl.ds(ch0 * CHM, CHM)]],
                                  xr0, sg0)
            w0 = pltpu.async_copy(wf_hbm.at[pl.ds(ebase + ch0 * CHM, CHM)],
                                  wf0, sw0)
            g1 = pltpu.async_copy(xl_hbm.at[src_v.at[pl.ds(ch1 * CHM, CHM)]],
                                  xr1, sg1)
            w1 = pltpu.async_copy(wf_hbm.at[pl.ds(ch1 * CHM + ebase, CHM)],
                                  wf1, sw1)
            pltpu.sync_copy(dst_hbm.at[pl.ds(ebase + ch0 * CHM, CHM)], dc0)
            pltpu.sync_copy(dst_hbm.at[pl.ds(ebase + ch1 * CHM, CHM)], dc1)
            g0.wait()
            w0.wait()
            mul(xr0, wf0)
            s0 = pltpu.async_copy(xr0, acc_sh.at[dc0], ss0, add=True)
            g1.wait()
            w1.wait()
            mul(xr1, wf1)
            s1 = pltpu.async_copy(xr1, acc_sh.at[dc1], ss1, add=True)
            s0.wait()
            s1.wait()
            return carry

        lax.fori_loop(0, npairs, pair, 0)

    @pl.when(cid == 0)
    def _():
        eb = sid * T0
        pltpu.sync_copy(src_hbm.at[pl.ds(eb, T0)], src_v.at[pl.ds(0, T0)])
        plsc.subcore_barrier()
        run(eb, T0 // (2 * CHM))

    @pl.when(cid == 1)
    def _():
        eb = E_C0 + sid * T1
        pltpu.sync_copy(src_hbm.at[pl.ds(eb, T1)], src_v.at[pl.ds(0, T1)])
        plsc.subcore_barrier()
        run(eb, T1 // (2 * CHM))

    plsc.subcore_barrier()
    pltpu.sync_copy(acc_sh.at[pl.ds(r0, rpt)],
                    agg_out.at[cid].at[pl.ds(r0, rpt)])


_sc_msgpass = functools.partial(
    pl.kernel,
    out_type=jax.ShapeDtypeStruct((2, N_PAD, H), jnp.float32),
    mesh=plsc.VectorSubcoreMesh(core_axis_name="c", subcore_axis_name="s"),
    scratch_types=[
        pltpu.VMEM((T0,), jnp.int32),
        pltpu.VMEM((CHM,), jnp.int32),
        pltpu.VMEM((CHM,), jnp.int32),
        pltpu.VMEM((CHM, H), jnp.float32),
        pltpu.VMEM((CHM, H), jnp.float32),
        pltpu.VMEM((CHM, H), jnp.float32),
        pltpu.VMEM((CHM, H), jnp.float32),
        pltpu.VMEM_SHARED((N_PAD, H), jnp.float32),
        pltpu.SemaphoreType.DMA,
        pltpu.SemaphoreType.DMA,
        pltpu.SemaphoreType.DMA,
        pltpu.SemaphoreType.DMA,
        pltpu.SemaphoreType.DMA,
        pltpu.SemaphoreType.DMA,
    ],
)(_sc_msgpass_body)


EROWS = E_PAD // H
BP = 256


def _tc_cdist_body(d2_ref, dist_ref, c_ref):
    d2 = d2_ref[...]
    dist = jnp.sqrt(d2 + 1e-12)
    c = 0.5 * (jnp.cos(dist * (np.pi / CUTOFF)) + 1.0)
    rows = (pl.program_id(0) * BP
            + lax.broadcasted_iota(jnp.int32, (BP, 1), 0))
    c = jnp.where(rows < E // H, c, 0.0)
    dist_ref[...] = dist
    c_ref[...] = c


def _tc_cdist(d2pk):
    spec = pl.BlockSpec((BP, H), lambda i: (i, 0))
    return pl.pallas_call(
        _tc_cdist_body,
        grid=(EROWS // BP,),
        in_specs=[spec],
        out_specs=[spec, spec],
        out_shape=[jax.ShapeDtypeStruct((EROWS, H), jnp.float32),
                   jax.ShapeDtypeStruct((EROWS, H), jnp.float32)],
    )(d2pk)


def _tc_wf_body(dist_ref, c_ref, w1, b1, w2, b2, wf_ref):
    dist = dist_ref[...]
    lane = lax.broadcasted_iota(jnp.int32, (1, 16), 1)
    offs = jnp.where(lane < NGAUSS, lane.astype(jnp.float32) * _DELTA, 1e4)
    diff = dist - offs
    rbf = jnp.exp(_COEFF * (diff * diff))
    t = _ssp(jnp.dot(rbf, w1[...], preferred_element_type=jnp.float32)
             + b1[...])
    wf_ref[...] = (jnp.dot(t, w2[...], preferred_element_type=jnp.float32)
                   + b2[...]) * c_ref[...]


def _tc_wf(dist_col, c_col, w1, b1, w2, b2):
    g = E_PAD // BE
    wspec = lambda shp: pl.BlockSpec(shp, lambda i: (0, 0))
    return pl.pallas_call(
        _tc_wf_body,
        grid=(g,),
        in_specs=[
            pl.BlockSpec((BE, 1), lambda i: (i, 0)),
            pl.BlockSpec((BE, 1), lambda i: (i, 0)),
            wspec((16, H)), wspec((1, H)), wspec((H, H)), wspec((1, H)),
        ],
        out_specs=pl.BlockSpec((BE, H), lambda i: (i, 0)),
        out_shape=jax.ShapeDtypeStruct((E_PAD, H), jnp.float32),
    )(dist_col, c_col, w1, b1, w2, b2)


def _tc_xl_body(h_ref, w_ref, o_ref):
    o_ref[...] = jnp.dot(h_ref[...], w_ref[...],
                         preferred_element_type=jnp.float32)


def _tc_xl(h, w):
    return pl.pallas_call(
        _tc_xl_body,
        grid=(N_PAD // BN,),
        in_specs=[pl.BlockSpec((BN, H), lambda i: (i, 0)),
                  pl.BlockSpec((H, H), lambda i: (0, 0))],
        out_specs=pl.BlockSpec((BN, H), lambda i: (i, 0)),
        out_shape=jax.ShapeDtypeStruct((N_PAD, H), jnp.float32),
    )(h, w)


def _tc_update_body(p0_ref, p1_ref, h_ref, l2w, l2b, lw, lb, l1wn,
                    hn_ref, xln_ref):
    agg = p0_ref[...] + p1_ref[...]
    t = _ssp(jnp.dot(agg, l2w[...], preferred_element_type=jnp.float32)
             + l2b[...])
    y = jnp.dot(t, lw[...], preferred_element_type=jnp.float32) + lb[...]
    hn = h_ref[...] + y
    hn_ref[...] = hn
    xln_ref[...] = jnp.dot(hn, l1wn[...], preferred_element_type=jnp.float32)


def _tc_update(p0, p1, h, l2w, l2b, lw, lb, l1wn):
    wspec = lambda shp: pl.BlockSpec(shp, lambda i: (0, 0))
    return pl.pallas_call(
        _tc_update_body,
        grid=(N_PAD // BN,),
        in_specs=[pl.BlockSpec((BN, H), lambda i: (i, 0)),
                  pl.BlockSpec((BN, H), lambda i: (i, 0)),
                  pl.BlockSpec((BN, H), lambda i: (i, 0)),
                  wspec((H, H)), wspec((1, H)), wspec((H, H)), wspec((1, H)),
                  wspec((H, H))],
        out_specs=[pl.BlockSpec((BN, H), lambda i: (i, 0)),
                   pl.BlockSpec((BN, H), lambda i: (i, 0))],
        out_shape=[jax.ShapeDtypeStruct((N_PAD, H), jnp.float32),
                   jax.ShapeDtypeStruct((N_PAD, H), jnp.float32)],
    )(p0, p1, h, l2w, l2b, lw, lb, l1wn)


def _tc_readout_body(h2p_ref, fl1b, fl2w, fl2b, batch_ref, pw, pb,
                     out_ref, g_ref):
    t = _ssp(h2p_ref[...] + fl1b[...])
    h2 = jnp.dot(t, fl2w[...], preferred_element_type=jnp.float32) + fl2b[...]
    bt = batch_ref[...]
    oh = (bt == lax.broadcasted_iota(jnp.int32, (1, NGRAPH), 1))
    oh = oh.astype(jnp.float32)
    g_part = lax.dot_general(oh, h2, (((0,), (0,)), ((), ())),
                             preferred_element_type=jnp.float32)
    pid = pl.program_id(0)

    @pl.when(pid == 0)
    def _():
        g_ref[...] = g_part

    @pl.when(pid > 0)
    def _():
        g_ref[...] = g_ref[...] + g_part

    @pl.when(pid == pl.num_programs(0) - 1)
    def _():
        out_ref[...] = (jnp.dot(g_ref[...], pw[...],
                                preferred_element_type=jnp.float32) + pb[...])


def _tc_readout(h2p, fl1b, fl2w, fl2b, batch2d, pw, pb):
    wspec = lambda shp: pl.BlockSpec(shp, lambda i: (0, 0))
    return pl.pallas_call(
        _tc_readout_body,
        grid=(N_PAD // BN,),
        in_specs=[pl.BlockSpec((BN, H), lambda i: (i, 0)),
                  wspec((1, H)), wspec((H, H)), wspec((1, H)),
                  pl.BlockSpec((BN, 1), lambda i: (i, 0)),
                  wspec((H, 1)), wspec((1, 1))],
        out_specs=pl.BlockSpec((NGRAPH, 1), lambda i: (0, 0)),
        out_shape=jax.ShapeDtypeStruct((NGRAPH, 1), jnp.float32),
        scratch_shapes=[pltpu.VMEM((NGRAPH, H), jnp.float32)],
    )(h2p, fl1b, fl2w, fl2b, batch2d, pw, pb)


def kernel(z, pos, batch, edge_index, emb, mw1_0, mb1_0, mw2_0, mb2_0, l1w_0,
           l2w_0, l2b_0, lw_0, lb_0, mw1_1, mb1_1, mw2_1, mb2_1, l1w_1,
           l2w_1, l2b_1, lw_1, lb_1, fl1w, fl1b, fl2w, fl2b, pw, pb):
    src = edge_index[0].astype(jnp.int32)
    dst = edge_index[1].astype(jnp.int32)
    epad = E_PAD - E
    src_p = jnp.concatenate([src, jnp.zeros((epad,), jnp.int32)])
    dst_p = jnp.concatenate([dst, jnp.zeros((epad,), jnp.int32)])
    z_p = jnp.concatenate([z.astype(jnp.int32),
                           jnp.zeros((N_PAD - N,), jnp.int32)])
    batch_p = jnp.concatenate([batch.astype(jnp.int32),
                               jnp.full((N_PAD - N,), NGRAPH, jnp.int32)])
    zeros_tbl = jnp.zeros((N_PAD, H), jnp.float32)

    mw1p0 = jnp.zeros((16, H), jnp.float32).at[:NGAUSS].set(mw1_0)
    mw1p1 = jnp.zeros((16, H), jnp.float32).at[:NGAUSS].set(mw1_1)
    fl1wp = jnp.zeros((H, H), jnp.float32).at[:, :H // 2].set(fl1w)
    fl1bp = jnp.zeros((1, H), jnp.float32).at[0, :H // 2].set(fl1b)
    fl2wp = jnp.zeros((H, H), jnp.float32).at[:H // 2].set(fl2w)
    r2 = lambda v: v.reshape(1, -1)

    dist2, h0 = _sc_prep(pos[:, 0], pos[:, 1], pos[:, 2], src_p, dst_p,
                         z_p, emb)
    dist_pk, c_pk = _tc_cdist(dist2.reshape(EROWS, H))
    dist_col = dist_pk.reshape(E_PAD, 1)
    c_col = c_pk.reshape(E_PAD, 1)
    wf0 = _tc_wf(dist_col, c_col, mw1p0, r2(mb1_0), mw2_0, r2(mb2_0))

    xl0 = _tc_xl(h0, l1w_0)
    agg0 = _sc_msgpass(xl0, wf0, src_p, dst_p, zeros_tbl)
    wf1 = _tc_wf(dist_col, c_col, mw1p1, r2(mb1_1), mw2_1, r2(mb2_1))
    h1, xl1 = _tc_update(agg0[0], agg0[1], h0, l2w_0, r2(l2b_0), lw_0,
                         r2(lb_0), l1w_1)

    agg1 = _sc_msgpass(xl1, wf1, src_p, dst_p, zeros_tbl)
    h2, h2p = _tc_update(agg1[0], agg1[1], h1, l2w_1, r2(l2b_1), lw_1,
                         r2(lb_1), fl1wp)
    del h2

    out = _tc_readout(h2p, fl1bp, fl2wp, r2(fl2b), batch_p.reshape(N_PAD, 1),
                      pw, r2(pb))
    return out


# speedup vs baseline: 3.2465x; 1.2885x over previous
import functools

import jax
import jax.numpy as jnp
import numpy as np
from jax import lax
from jax.experimental import pallas as pl
from jax.experimental.pallas import tpu as pltpu
from jax.experimental.pallas import tpu_sc as plsc

N = 10000
E = 320000
H = 128
NGAUSS = 10
NGRAPH = 64
CUTOFF = 10.0

NW = 32
EPT = 10240
E_PAD = EPT * NW
CH = 128
NPT = 320
N_PAD = NPT * NW
BE = 1024
BN = 1024

_LOG2 = float(np.log(2.0))
_DELTA = CUTOFF / (NGAUSS - 1)
_COEFF = -0.5 / (_DELTA * _DELTA)
_OFFS = np.full((1, 16), 1e4, np.float32)
_OFFS[0, :NGAUSS] = np.linspace(0.0, CUTOFF, NGAUSS, dtype=np.float32)


def _ssp(x):
    return jnp.maximum(x, 0.0) + jnp.log1p(jnp.exp(-jnp.abs(x))) - _LOG2


PT0 = 12800
PT1 = 7680

def _sc_prep_body(px_hbm, py_hbm, pz_hbm, src_hbm, dst_hbm, z_hbm, emb_hbm,
                  d2_out, h0_out,
                  src_v, dst_v, sx_v, sy_v, sz_v, tx_v, ty_v, tz_v,
                  ux_v, uy_v, uz_v, vx_v, vy_v, vz_v,
                  d2_v, z_v, emb_v, sem, sem2):
    cid = lax.axis_index("c")
    sid = lax.axis_index("s")
    wid = cid * 16 + sid
    ebase = jnp.where(cid == 0, sid * PT0, PT0 * 16 + sid * PT1)
    ept_c = jnp.where(cid == 0, PT0, PT1)

    def fire(c, bufs, sem_c):
        isrc = src_v.at[pl.ds(c * CH, CH)]
        idst = dst_v.at[pl.ds(c * CH, CH)]
        return [pltpu.async_copy(px_hbm.at[isrc], bufs[0], sem_c),
                pltpu.async_copy(py_hbm.at[isrc], bufs[1], sem_c),
                pltpu.async_copy(pz_hbm.at[isrc], bufs[2], sem_c),
                pltpu.async_copy(px_hbm.at[idst], bufs[3], sem_c),
                pltpu.async_copy(py_hbm.at[idst], bufs[4], sem_c),
                pltpu.async_copy(pz_hbm.at[idst], bufs[5], sem_c)]

    bufs_a = (sx_v, sy_v, sz_v, tx_v, ty_v, tz_v)
    bufs_b = (ux_v, uy_v, uz_v, vx_v, vy_v, vz_v)

    def pair(k, carry):
        cps_a = fire(2 * k, bufs_a, sem)
        cps_b = fire(2 * k + 1, bufs_b, sem2)
        for cp in cps_a:
            cp.wait()
        for c, bufs in ((2 * k, bufs_a), (2 * k + 1, bufs_b)):
            if bufs is bufs_b:
                for cp in cps_b:
                    cp.wait()
            for v in range(CH // 16):
                sl = pl.ds(v * 16, 16)
                ddx = bufs[0][sl] - bufs[3][sl]
                ddy = bufs[1][sl] - bufs[4][sl]
                ddz = bufs[2][sl] - bufs[5][sl]
                d2_v[pl.ds(c * CH + v * 16, 16)] = (
                    ddx * ddx + ddy * ddy + ddz * ddz)
        return carry

    @pl.when(cid == 0)
    def _():
        pltpu.sync_copy(src_hbm.at[pl.ds(sid * PT0, PT0)],
                        src_v.at[pl.ds(0, PT0)])
        pltpu.sync_copy(dst_hbm.at[pl.ds(sid * PT0, PT0)],
                        dst_v.at[pl.ds(0, PT0)])
        lax.fori_loop(0, PT0 // CH // 2, pair, 0)
        pltpu.sync_copy(d2_v.at[pl.ds(0, PT0)],
                        d2_out.at[pl.ds(sid * PT0, PT0)])

    @pl.when(cid == 1)
    def _():
        eb = PT0 * 16 + sid * PT1
        pltpu.sync_copy(src_hbm.at[pl.ds(eb, PT1)], src_v.at[pl.ds(0, PT1)])
        pltpu.sync_copy(dst_hbm.at[pl.ds(eb, PT1)], dst_v.at[pl.ds(0, PT1)])
        lax.fori_loop(0, PT1 // CH // 2, pair, 0)
        pltpu.sync_copy(d2_v.at[pl.ds(0, PT1)], d2_out.at[pl.ds(eb, PT1)])

    nbase = wid * NPT
    pltpu.sync_copy(z_hbm.at[pl.ds(nbase, NPT)], z_v)
    for c0 in range(0, NPT, 80):
        pltpu.async_copy(emb_hbm.at[z_v.at[pl.ds(c0, 80)]],
                         emb_v.at[pl.ds(c0, 80)], sem).wait()
    pltpu.sync_copy(emb_v, h0_out.at[pl.ds(nbase, NPT)])


_sc_prep = functools.partial(
    pl.kernel,
    out_type=[jax.ShapeDtypeStruct((E_PAD,), jnp.float32),
              jax.ShapeDtypeStruct((N_PAD, H), jnp.float32)],
    mesh=plsc.VectorSubcoreMesh(core_axis_name="c", subcore_axis_name="s"),
    scratch_types=(
        [pltpu.VMEM((PT0,), jnp.int32),
         pltpu.VMEM((PT0,), jnp.int32)]
        + [pltpu.VMEM((CH,), jnp.float32) for _ in range(12)]
        + [pltpu.VMEM((PT0,), jnp.float32),
           pltpu.VMEM((NPT,), jnp.int32),
           pltpu.VMEM((NPT, H), jnp.float32),
           pltpu.SemaphoreType.DMA,
           pltpu.SemaphoreType.DMA]
    ),
)(_sc_prep_body)


CHM = 64
E_C0 = 245760
E_C1 = E_PAD - E_C0
T0 = E_C0 // 16
T1 = E_C1 // 16


def _sc_msgpass_body(xl_hbm, wf_hbm, src_hbm, dst_hbm, zeros_hbm,
                     agg_out,
                     src_v, dc0, dc1, xr0, xr1, wf0, wf1, acc_sh,
                     sg0, sg1, sw0, sw1, ss0, ss1):
    cid = lax.axis_index("c")
    sid = lax.axis_index("s")
    rpt = N_PAD // 16
    r0 = sid * rpt
    pltpu.sync_copy(zeros_hbm.at[pl.ds(r0, rpt)], acc_sh.at[pl.ds(r0, rpt)])

    def mul(xr, wf):
        def row(r, c2):
            for v in range(8):
                sl = pl.ds(v * 16, 16)
                xr[r, sl] = xr[r, sl] * wf[r, sl]
            return c2

        lax.fori_loop(0, CHM, row, 0)

    def run(ebase, npairs):
        def pair(k, carry):
            ch0 = 2 * k
            ch1 = 2 * k + 1
            g0 = pltpu.async_copy(xl_hbm.at[src_v.at[pl.ds(ch0 * CHM, CHM)]],
                                  xr0, sg0)
            w0 = pltpu.async_copy(wf_hbm.at[pl.ds(ebase + ch0 * CHM, CHM)],
                                  wf0, sw0)
            g1 = pltpu.async_copy(xl_hbm.at[src_v.at[pl.ds(ch1 * CHM, CHM)]],
                                  xr1, sg1)
            w1 = pltpu.async_copy(wf_hbm.at[pl.ds(ch1 * CHM + ebase, CHM)],
                                  wf1, sw1)
            pltpu.sync_copy(dst_hbm.at[pl.ds(ebase + ch0 * CHM, CHM)], dc0)
            pltpu.sync_copy(dst_hbm.at[pl.ds(ebase + ch1 * CHM, CHM)], dc1)
            g0.wait()
            w0.wait()
            mul(xr0, wf0)
            s0 = pltpu.async_copy(xr0, acc_sh.at[dc0], ss0, add=True)
            g1.wait()
            w1.wait()
            mul(xr1, wf1)
            s1 = pltpu.async_copy(xr1, acc_sh.at[dc1], ss1, add=True)
            s0.wait()
            s1.wait()
            return carry

        lax.fori_loop(0, npairs, pair, 0)

    @pl.when(cid == 0)
    def _():
        eb = sid * T0
        pltpu.sync_copy(src_hbm.at[pl.ds(eb, T0)], src_v.at[pl.ds(0, T0)])
        plsc.subcore_barrier()
        run(eb, T0 // (2 * CHM))

    @pl.when(cid == 1)
    def _():
        eb = E_C0 + sid * T1
        pltpu.sync_copy(src_hbm.at[pl.ds(eb, T1)], src_v.at[pl.ds(0, T1)])
        plsc.subcore_barrier()
        run(eb, T1 // (2 * CHM))

    plsc.subcore_barrier()
    pltpu.sync_copy(acc_sh.at[pl.ds(r0, rpt)],
                    agg_out.at[cid].at[pl.ds(r0, rpt)])


_sc_msgpass = functools.partial(
    pl.kernel,
    out_type=jax.ShapeDtypeStruct((2, N_PAD, H), jnp.float32),
    mesh=plsc.VectorSubcoreMesh(core_axis_name="c", subcore_axis_name="s"),
    scratch_types=[
        pltpu.VMEM((T0,), jnp.int32),
        pltpu.VMEM((CHM,), jnp.int32),
        pltpu.VMEM((CHM,), jnp.int32),
        pltpu.VMEM((CHM, H), jnp.float32),
        pltpu.VMEM((CHM, H), jnp.float32),
        pltpu.VMEM((CHM, H), jnp.float32),
        pltpu.VMEM((CHM, H), jnp.float32),
        pltpu.VMEM_SHARED((N_PAD, H), jnp.float32),
        pltpu.SemaphoreType.DMA,
        pltpu.SemaphoreType.DMA,
        pltpu.SemaphoreType.DMA,
        pltpu.SemaphoreType.DMA,
        pltpu.SemaphoreType.DMA,
        pltpu.SemaphoreType.DMA,
    ],
)(_sc_msgpass_body)


EROWS = E_PAD // H
BP = 256


def _tc_cdist_body(d2_ref, dist_ref, c_ref):
    d2 = d2_ref[...]
    dist = jnp.sqrt(d2 + 1e-12)
    c = 0.5 * (jnp.cos(dist * (np.pi / CUTOFF)) + 1.0)
    rows = (pl.program_id(0) * BP
            + lax.broadcasted_iota(jnp.int32, (BP, 1), 0))
    c = jnp.where(rows < E // H, c, 0.0)
    dist_ref[...] = dist
    c_ref[...] = c


def _tc_cdist(d2pk):
    spec = pl.BlockSpec((BP, H), lambda i: (i, 0))
    return pl.pallas_call(
        _tc_cdist_body,
        grid=(EROWS // BP,),
        in_specs=[spec],
        out_specs=[spec, spec],
        out_shape=[jax.ShapeDtypeStruct((EROWS, H), jnp.float32),
                   jax.ShapeDtypeStruct((EROWS, H), jnp.float32)],
    )(d2pk)


def _tc_wf_body(dist_ref, c_ref, w1, b1, w2, b2, wf_ref):
    idr = lax.broadcasted_iota(jnp.int32, (H, H), 0)
    idc = lax.broadcasted_iota(jnp.int32, (H, H), 1)
    ident = (idr == idc).astype(jnp.float32)
    kio = lax.broadcasted_iota(jnp.int32, (16, 1), 0)
    offs = jnp.where(kio < NGAUSS, kio.astype(jnp.float32) * _DELTA, 1e4)
    for s in range(BE // H):
        drow = dist_ref[s:s + 1, :]
        diff = jnp.broadcast_to(drow, (16, H)) - offs
        rbf = jnp.exp(_COEFF * (diff * diff))
        t = lax.dot_general(rbf, w1[...], (((0,), (0,)), ((), ())),
                            preferred_element_type=jnp.float32)
        t = _ssp(t + b1[...])
        y = jnp.dot(t, w2[...], preferred_element_type=jnp.float32) + b2[...]
        crow = c_ref[s:s + 1, :]
        ccol = lax.dot_general(ident, crow, (((1,), (1,)), ((), ())),
                               preferred_element_type=jnp.float32)
        wf_ref[pl.ds(s * H, H), :] = y * ccol


def _tc_wf(dist_pk, c_pk, w1, b1, w2, b2):
    g = E_PAD // BE
    wspec = lambda shp: pl.BlockSpec(shp, lambda i: (0, 0))
    pspec = pl.BlockSpec((BE // H, H), lambda i: (i, 0))
    return pl.pallas_call(
        _tc_wf_body,
        grid=(g,),
        in_specs=[
            pspec, pspec,
            wspec((16, H)), wspec((1, H)), wspec((H, H)), wspec((1, H)),
        ],
        out_specs=pl.BlockSpec((BE, H), lambda i: (i, 0)),
        out_shape=jax.ShapeDtypeStruct((E_PAD, H), jnp.float32),
    )(dist_pk, c_pk, w1, b1, w2, b2)


def _tc_xl_body(h_ref, w_ref, o_ref):
    o_ref[...] = jnp.dot(h_ref[...], w_ref[...],
                         preferred_element_type=jnp.float32)


def _tc_xl(h, w):
    return pl.pallas_call(
        _tc_xl_body,
        grid=(N_PAD // BN,),
        in_specs=[pl.BlockSpec((BN, H), lambda i: (i, 0)),
                  pl.BlockSpec((H, H), lambda i: (0, 0))],
        out_specs=pl.BlockSpec((BN, H), lambda i: (i, 0)),
        out_shape=jax.ShapeDtypeStruct((N_PAD, H), jnp.float32),
    )(h, w)


def _tc_update_body(p0_ref, p1_ref, h_ref, l2w, l2b, lw, lb, l1wn,
                    hn_ref, xln_ref):
    agg = p0_ref[...] + p1_ref[...]
    t = _ssp(jnp.dot(agg, l2w[...], preferred_element_type=jnp.float32)
             + l2b[...])
    y = jnp.dot(t, lw[...], preferred_element_type=jnp.float32) + lb[...]
    hn = h_ref[...] + y
    hn_ref[...] = hn
    xln_ref[...] = jnp.dot(hn, l1wn[...], preferred_element_type=jnp.float32)


def _tc_update(p0, p1, h, l2w, l2b, lw, lb, l1wn):
    wspec = lambda shp: pl.BlockSpec(shp, lambda i: (0, 0))
    return pl.pallas_call(
        _tc_update_body,
        grid=(N_PAD // BN,),
        in_specs=[pl.BlockSpec((BN, H), lambda i: (i, 0)),
                  pl.BlockSpec((BN, H), lambda i: (i, 0)),
                  pl.BlockSpec((BN, H), lambda i: (i, 0)),
                  wspec((H, H)), wspec((1, H)), wspec((H, H)), wspec((1, H)),
                  wspec((H, H))],
        out_specs=[pl.BlockSpec((BN, H), lambda i: (i, 0)),
                   pl.BlockSpec((BN, H), lambda i: (i, 0))],
        out_shape=[jax.ShapeDtypeStruct((N_PAD, H), jnp.float32),
                   jax.ShapeDtypeStruct((N_PAD, H), jnp.float32)],
    )(p0, p1, h, l2w, l2b, lw, lb, l1wn)


def _tc_readout_body(h2p_ref, fl1b, fl2w, fl2b, batch_ref, pw, pb,
                     out_ref, g_ref):
    t = _ssp(h2p_ref[...] + fl1b[...])
    h2 = jnp.dot(t, fl2w[...], preferred_element_type=jnp.float32) + fl2b[...]
    bt = batch_ref[...]
    oh = (bt == lax.broadcasted_iota(jnp.int32, (1, NGRAPH), 1))
    oh = oh.astype(jnp.float32)
    g_part = lax.dot_general(oh, h2, (((0,), (0,)), ((), ())),
                             preferred_element_type=jnp.float32)
    pid = pl.program_id(0)

    @pl.when(pid == 0)
    def _():
        g_ref[...] = g_part

    @pl.when(pid > 0)
    def _():
        g_ref[...] = g_ref[...] + g_part

    @pl.when(pid == pl.num_programs(0) - 1)
    def _():
        out_ref[...] = (jnp.dot(g_ref[...], pw[...],
                                preferred_element_type=jnp.float32) + pb[...])


def _tc_readout(h2p, fl1b, fl2w, fl2b, batch2d, pw, pb):
    wspec = lambda shp: pl.BlockSpec(shp, lambda i: (0, 0))
    return pl.pallas_call(
        _tc_readout_body,
        grid=(N_PAD // BN,),
        in_specs=[pl.BlockSpec((BN, H), lambda i: (i, 0)),
                  wspec((1, H)), wspec((H, H)), wspec((1, H)),
                  pl.BlockSpec((BN, 1), lambda i: (i, 0)),
                  wspec((H, 1)), wspec((1, 1))],
        out_specs=pl.BlockSpec((NGRAPH, 1), lambda i: (0, 0)),
        out_shape=jax.ShapeDtypeStruct((NGRAPH, 1), jnp.float32),
        scratch_shapes=[pltpu.VMEM((NGRAPH, H), jnp.float32)],
    )(h2p, fl1b, fl2w, fl2b, batch2d, pw, pb)


def kernel(z, pos, batch, edge_index, emb, mw1_0, mb1_0, mw2_0, mb2_0, l1w_0,
           l2w_0, l2b_0, lw_0, lb_0, mw1_1, mb1_1, mw2_1, mb2_1, l1w_1,
           l2w_1, l2b_1, lw_1, lb_1, fl1w, fl1b, fl2w, fl2b, pw, pb):
    src = edge_index[0].astype(jnp.int32)
    dst = edge_index[1].astype(jnp.int32)
    epad = E_PAD - E
    src_p = jnp.concatenate([src, jnp.zeros((epad,), jnp.int32)])
    dst_p = jnp.concatenate([dst, jnp.zeros((epad,), jnp.int32)])
    z_p = jnp.concatenate([z.astype(jnp.int32),
                           jnp.zeros((N_PAD - N,), jnp.int32)])
    batch_p = jnp.concatenate([batch.astype(jnp.int32),
                               jnp.full((N_PAD - N,), NGRAPH, jnp.int32)])
    zeros_tbl = jnp.zeros((N_PAD, H), jnp.float32)

    mw1p0 = jnp.zeros((16, H), jnp.float32).at[:NGAUSS].set(mw1_0)
    mw1p1 = jnp.zeros((16, H), jnp.float32).at[:NGAUSS].set(mw1_1)
    fl1wp = jnp.zeros((H, H), jnp.float32).at[:, :H // 2].set(fl1w)
    fl1bp = jnp.zeros((1, H), jnp.float32).at[0, :H // 2].set(fl1b)
    fl2wp = jnp.zeros((H, H), jnp.float32).at[:H // 2].set(fl2w)
    r2 = lambda v: v.reshape(1, -1)

    dist2, h0 = _sc_prep(pos[:, 0], pos[:, 1], pos[:, 2], src_p, dst_p,
                         z_p, emb)
    dist_pk, c_pk = _tc_cdist(dist2.reshape(EROWS, H))
    wf0 = _tc_wf(dist_pk, c_pk, mw1p0, r2(mb1_0), mw2_0, r2(mb2_0))

    xl0 = _tc_xl(h0, l1w_0)
    agg0 = _sc_msgpass(xl0, wf0, src_p, dst_p, zeros_tbl)
    wf1 = _tc_wf(dist_pk, c_pk, mw1p1, r2(mb1_1), mw2_1, r2(mb2_1))
    h1, xl1 = _tc_update(agg0[0], agg0[1], h0, l2w_0, r2(l2b_0), lw_0,
                         r2(lb_0), l1w_1)

    agg1 = _sc_msgpass(xl1, wf1, src_p, dst_p, zeros_tbl)
    h2, h2p = _tc_update(agg1[0], agg1[1], h1, l2w_1, r2(l2b_1), lw_1,
                         r2(lb_1), fl1wp)
    del h2

    out = _tc_readout(h2p, fl1bp, fl2wp, r2(fl2b), batch_p.reshape(N_PAD, 1),
                      pw, r2(pb))
    return out


# speedup vs baseline: 3.2917x; 1.0139x over previous
import functools

import jax
import jax.numpy as jnp
import numpy as np
from jax import lax
from jax.experimental import pallas as pl
from jax.experimental.pallas import tpu as pltpu
from jax.experimental.pallas import tpu_sc as plsc

N = 10000
E = 320000
H = 128
NGAUSS = 10
NGRAPH = 64
CUTOFF = 10.0

NW = 32
EPT = 10240
E_PAD = EPT * NW
CH = 128
NPT = 320
N_PAD = NPT * NW
BE = 1024
BN = 1024

_LOG2 = float(np.log(2.0))
_DELTA = CUTOFF / (NGAUSS - 1)
_COEFF = -0.5 / (_DELTA * _DELTA)
_OFFS = np.full((1, 16), 1e4, np.float32)
_OFFS[0, :NGAUSS] = np.linspace(0.0, CUTOFF, NGAUSS, dtype=np.float32)


def _ssp(x):
    return jnp.maximum(x, 0.0) + jnp.log1p(jnp.exp(-jnp.abs(x))) - _LOG2


PT0 = 13824
PT1 = 6656

def _sc_prep_body(px_hbm, py_hbm, pz_hbm, src_hbm, dst_hbm, z_hbm, emb_hbm,
                  d2_out, h0_out,
                  src_v, dst_v, sx_v, sy_v, sz_v, tx_v, ty_v, tz_v,
                  ux_v, uy_v, uz_v, vx_v, vy_v, vz_v,
                  d2_v, z_v, emb_v, sem, sem2):
    cid = lax.axis_index("c")
    sid = lax.axis_index("s")
    wid = cid * 16 + sid
    ebase = jnp.where(cid == 0, sid * PT0, PT0 * 16 + sid * PT1)
    ept_c = jnp.where(cid == 0, PT0, PT1)

    def fire(c, bufs, sem_c):
        isrc = src_v.at[pl.ds(c * CH, CH)]
        idst = dst_v.at[pl.ds(c * CH, CH)]
        return [pltpu.async_copy(px_hbm.at[isrc], bufs[0], sem_c),
                pltpu.async_copy(py_hbm.at[isrc], bufs[1], sem_c),
                pltpu.async_copy(pz_hbm.at[isrc], bufs[2], sem_c),
                pltpu.async_copy(px_hbm.at[idst], bufs[3], sem_c),
                pltpu.async_copy(py_hbm.at[idst], bufs[4], sem_c),
                pltpu.async_copy(pz_hbm.at[idst], bufs[5], sem_c)]

    bufs_a = (sx_v, sy_v, sz_v, tx_v, ty_v, tz_v)
    bufs_b = (ux_v, uy_v, uz_v, vx_v, vy_v, vz_v)

    def pair(k, carry):
        cps_a = fire(2 * k, bufs_a, sem)
        cps_b = fire(2 * k + 1, bufs_b, sem2)
        for cp in cps_a:
            cp.wait()
        for c, bufs in ((2 * k, bufs_a), (2 * k + 1, bufs_b)):
            if bufs is bufs_b:
                for cp in cps_b:
                    cp.wait()
            for v in range(CH // 16):
                sl = pl.ds(v * 16, 16)
                ddx = bufs[0][sl] - bufs[3][sl]
                ddy = bufs[1][sl] - bufs[4][sl]
                ddz = bufs[2][sl] - bufs[5][sl]
                d2_v[pl.ds(c * CH + v * 16, 16)] = (
                    ddx * ddx + ddy * ddy + ddz * ddz)
        return carry

    @pl.when(cid == 0)
    def _():
        pltpu.sync_copy(src_hbm.at[pl.ds(sid * PT0, PT0)],
                        src_v.at[pl.ds(0, PT0)])
        pltpu.sync_copy(dst_hbm.at[pl.ds(sid * PT0, PT0)],
                        dst_v.at[pl.ds(0, PT0)])
        lax.fori_loop(0, PT0 // CH // 2, pair, 0)
        pltpu.sync_copy(d2_v.at[pl.ds(0, PT0)],
                        d2_out.at[pl.ds(sid * PT0, PT0)])

    @pl.when(cid == 1)
    def _():
        eb = PT0 * 16 + sid * PT1
        pltpu.sync_copy(src_hbm.at[pl.ds(eb, PT1)], src_v.at[pl.ds(0, PT1)])
        pltpu.sync_copy(dst_hbm.at[pl.ds(eb, PT1)], dst_v.at[pl.ds(0, PT1)])
        lax.fori_loop(0, PT1 // CH // 2, pair, 0)
        pltpu.sync_copy(d2_v.at[pl.ds(0, PT1)], d2_out.at[pl.ds(eb, PT1)])

    nbase = wid * NPT
    pltpu.sync_copy(z_hbm.at[pl.ds(nbase, NPT)], z_v)
    for c0 in range(0, NPT, 80):
        pltpu.async_copy(emb_hbm.at[z_v.at[pl.ds(c0, 80)]],
                         emb_v.at[pl.ds(c0, 80)], sem).wait()
    pltpu.sync_copy(emb_v, h0_out.at[pl.ds(nbase, NPT)])


_sc_prep = functools.partial(
    pl.kernel,
    out_type=[jax.ShapeDtypeStruct((E_PAD,), jnp.float32),
              jax.ShapeDtypeStruct((N_PAD, H), jnp.float32)],
    mesh=plsc.VectorSubcoreMesh(core_axis_name="c", subcore_axis_name="s"),
    scratch_types=(
        [pltpu.VMEM((PT0,), jnp.int32),
         pltpu.VMEM((PT0,), jnp.int32)]
        + [pltpu.VMEM((CH,), jnp.float32) for _ in range(12)]
        + [pltpu.VMEM((PT0,), jnp.float32),
           pltpu.VMEM((NPT,), jnp.int32),
           pltpu.VMEM((NPT, H), jnp.float32),
           pltpu.SemaphoreType.DMA,
           pltpu.SemaphoreType.DMA]
    ),
)(_sc_prep_body)


CHM = 64
E_C0 = 253952
E_C1 = E_PAD - E_C0
T0 = E_C0 // 16
T1 = E_C1 // 16


def _sc_msgpass_body(xl_hbm, wf_hbm, src_hbm, dst_hbm, zeros_hbm,
                     agg_out,
                     src_v, dc0, dc1, xr0, xr1, wf0, wf1, acc_sh,
                     sg0, sg1, sw0, sw1, ss0, ss1):
    cid = lax.axis_index("c")
    sid = lax.axis_index("s")
    rpt = N_PAD // 16
    r0 = sid * rpt
    pltpu.sync_copy(zeros_hbm.at[pl.ds(r0, rpt)], acc_sh.at[pl.ds(r0, rpt)])

    def mul(xr, wf):
        def row(r, c2):
            for v in range(8):
                sl = pl.ds(v * 16, 16)
                xr[r, sl] = xr[r, sl] * wf[r, sl]
            return c2

        lax.fori_loop(0, CHM, row, 0)

    def run(ebase, npairs):
        def pair(k, carry):
            ch0 = 2 * k
            ch1 = 2 * k + 1
            g0 = pltpu.async_copy(xl_hbm.at[src_v.at[pl.ds(ch0 * CHM, CHM)]],
                                  xr0, sg0)
            w0 = pltpu.async_copy(wf_hbm.at[pl.ds(ebase + ch0 * CHM, CHM)],
                                  wf0, sw0)
            g1 = pltpu.async_copy(xl_hbm.at[src_v.at[pl.ds(ch1 * CHM, CHM)]],
                                  xr1, sg1)
            w1 = pltpu.async_copy(wf_hbm.at[pl.ds(ch1 * CHM + ebase, CHM)],
                                  wf1, sw1)
            pltpu.sync_copy(dst_hbm.at[pl.ds(ebase + ch0 * CHM, CHM)], dc0)
            pltpu.sync_copy(dst_hbm.at[pl.ds(ebase + ch1 * CHM, CHM)], dc1)
            g0.wait()
            w0.wait()
            mul(xr0, wf0)
            s0 = pltpu.async_copy(xr0, acc_sh.at[dc0], ss0, add=True)
            g1.wait()
            w1.wait()
            mul(xr1, wf1)
            s1 = pltpu.async_copy(xr1, acc_sh.at[dc1], ss1, add=True)
            s0.wait()
            s1.wait()
            return carry

        lax.fori_loop(0, npairs, pair, 0)

    @pl.when(cid == 0)
    def _():
        eb = sid * T0
        pltpu.sync_copy(src_hbm.at[pl.ds(eb, T0)], src_v.at[pl.ds(0, T0)])
        plsc.subcore_barrier()
        run(eb, T0 // (2 * CHM))

    @pl.when(cid == 1)
    def _():
        eb = E_C0 + sid * T1
        pltpu.sync_copy(src_hbm.at[pl.ds(eb, T1)], src_v.at[pl.ds(0, T1)])
        plsc.subcore_barrier()
        run(eb, T1 // (2 * CHM))

    plsc.subcore_barrier()
    pltpu.sync_copy(acc_sh.at[pl.ds(r0, rpt)],
                    agg_out.at[cid].at[pl.ds(r0, rpt)])


_sc_msgpass = functools.partial(
    pl.kernel,
    out_type=jax.ShapeDtypeStruct((2, N_PAD, H), jnp.float32),
    mesh=plsc.VectorSubcoreMesh(core_axis_name="c", subcore_axis_name="s"),
    scratch_types=[
        pltpu.VMEM((T0,), jnp.int32),
        pltpu.VMEM((CHM,), jnp.int32),
        pltpu.VMEM((CHM,), jnp.int32),
        pltpu.VMEM((CHM, H), jnp.float32),
        pltpu.VMEM((CHM, H), jnp.float32),
        pltpu.VMEM((CHM, H), jnp.float32),
        pltpu.VMEM((CHM, H), jnp.float32),
        pltpu.VMEM_SHARED((N_PAD, H), jnp.float32),
        pltpu.SemaphoreType.DMA,
        pltpu.SemaphoreType.DMA,
        pltpu.SemaphoreType.DMA,
        pltpu.SemaphoreType.DMA,
        pltpu.SemaphoreType.DMA,
        pltpu.SemaphoreType.DMA,
    ],
)(_sc_msgpass_body)


EROWS = E_PAD // H
BP = 256


def _tc_cdist_body(d2_ref, dist_ref, c_ref):
    d2 = d2_ref[...]
    dist = jnp.sqrt(d2 + 1e-12)
    c = 0.5 * (jnp.cos(dist * (np.pi / CUTOFF)) + 1.0)
    rows = (pl.program_id(0) * BP
            + lax.broadcasted_iota(jnp.int32, (BP, 1), 0))
    c = jnp.where(rows < E // H, c, 0.0)
    dist_ref[...] = dist
    c_ref[...] = c


def _tc_cdist(d2pk):
    spec = pl.BlockSpec((BP, H), lambda i: (i, 0))
    return pl.pallas_call(
        _tc_cdist_body,
        grid=(EROWS // BP,),
        in_specs=[spec],
        out_specs=[spec, spec],
        out_shape=[jax.ShapeDtypeStruct((EROWS, H), jnp.float32),
                   jax.ShapeDtypeStruct((EROWS, H), jnp.float32)],
    )(d2pk)


def _tc_wf_body(dist_ref, c_ref, w1, b1, w2, b2, wf_ref):
    idr = lax.broadcasted_iota(jnp.int32, (H, H), 0)
    idc = lax.broadcasted_iota(jnp.int32, (H, H), 1)
    ident = (idr == idc).astype(jnp.float32)
    kio = lax.broadcasted_iota(jnp.int32, (16, 1), 0)
    offs = jnp.where(kio < NGAUSS, kio.astype(jnp.float32) * _DELTA, 1e4)
    for s in range(BE // H):
        drow = dist_ref[s:s + 1, :]
        diff = jnp.broadcast_to(drow, (16, H)) - offs
        rbf = jnp.exp(_COEFF * (diff * diff))
        t = lax.dot_general(rbf, w1[...], (((0,), (0,)), ((), ())),
                            preferred_element_type=jnp.float32)
        t = _ssp(t + b1[...])
        y = jnp.dot(t, w2[...], preferred_element_type=jnp.float32) + b2[...]
        crow = c_ref[s:s + 1, :]
        ccol = lax.dot_general(ident, crow, (((1,), (1,)), ((), ())),
                               preferred_element_type=jnp.float32)
        wf_ref[pl.ds(s * H, H), :] = y * ccol


def _tc_wf(dist_pk, c_pk, w1, b1, w2, b2):
    g = E_PAD // BE
    wspec = lambda shp: pl.BlockSpec(shp, lambda i: (0, 0))
    pspec = pl.BlockSpec((BE // H, H), lambda i: (i, 0))
    return pl.pallas_call(
        _tc_wf_body,
        grid=(g,),
        in_specs=[
            pspec, pspec,
            wspec((16, H)), wspec((1, H)), wspec((H, H)), wspec((1, H)),
        ],
        out_specs=pl.BlockSpec((BE, H), lambda i: (i, 0)),
        out_shape=jax.ShapeDtypeStruct((E_PAD, H), jnp.float32),
    )(dist_pk, c_pk, w1, b1, w2, b2)


def _tc_xl_body(h_ref, w_ref, o_ref):
    o_ref[...] = jnp.dot(h_ref[...], w_ref[...],
                         preferred_element_type=jnp.float32)


def _tc_xl(h, w):
    return pl.pallas_call(
        _tc_xl_body,
        grid=(N_PAD // BN,),
        in_specs=[pl.BlockSpec((BN, H), lambda i: (i, 0)),
                  pl.BlockSpec((H, H), lambda i: (0, 0))],
        out_specs=pl.BlockSpec((BN, H), lambda i: (i, 0)),
        out_shape=jax.ShapeDtypeStruct((N_PAD, H), jnp.float32),
    )(h, w)


def _tc_update_body(p0_ref, p1_ref, h_ref, l2w, l2b, lw, lb, l1wn,
                    hn_ref, xln_ref):
    agg = p0_ref[...] + p1_ref[...]
    t = _ssp(jnp.dot(agg, l2w[...], preferred_element_type=jnp.float32)
             + l2b[...])
    y = jnp.dot(t, lw[...], preferred_element_type=jnp.float32) + lb[...]
    hn = h_ref[...] + y
    hn_ref[...] = hn
    xln_ref[...] = jnp.dot(hn, l1wn[...], preferred_element_type=jnp.float32)


def _tc_update(p0, p1, h, l2w, l2b, lw, lb, l1wn):
    wspec = lambda shp: pl.BlockSpec(shp, lambda i: (0, 0))
    return pl.pallas_call(
        _tc_update_body,
        grid=(N_PAD // BN,),
        in_specs=[pl.BlockSpec((BN, H), lambda i: (i, 0)),
                  pl.BlockSpec((BN, H), lambda i: (i, 0)),
                  pl.BlockSpec((BN, H), lambda i: (i, 0)),
                  wspec((H, H)), wspec((1, H)), wspec((H, H)), wspec((1, H)),
                  wspec((H, H))],
        out_specs=[pl.BlockSpec((BN, H), lambda i: (i, 0)),
                   pl.BlockSpec((BN, H), lambda i: (i, 0))],
        out_shape=[jax.ShapeDtypeStruct((N_PAD, H), jnp.float32),
                   jax.ShapeDtypeStruct((N_PAD, H), jnp.float32)],
    )(p0, p1, h, l2w, l2b, lw, lb, l1wn)


def _tc_readout_body(h2p_ref, fl1b, fl2w, fl2b, batch_ref, pw, pb,
                     out_ref, g_ref):
    t = _ssp(h2p_ref[...] + fl1b[...])
    h2 = jnp.dot(t, fl2w[...], preferred_element_type=jnp.float32) + fl2b[...]
    bt = batch_ref[...]
    oh = (bt == lax.broadcasted_iota(jnp.int32, (1, NGRAPH), 1))
    oh = oh.astype(jnp.float32)
    g_part = lax.dot_general(oh, h2, (((0,), (0,)), ((), ())),
                             preferred_element_type=jnp.float32)
    pid = pl.program_id(0)

    @pl.when(pid == 0)
    def _():
        g_ref[...] = g_part

    @pl.when(pid > 0)
    def _():
        g_ref[...] = g_ref[...] + g_part

    @pl.when(pid == pl.num_programs(0) - 1)
    def _():
        out_ref[...] = (jnp.dot(g_ref[...], pw[...],
                                preferred_element_type=jnp.float32) + pb[...])


def _tc_readout(h2p, fl1b, fl2w, fl2b, batch2d, pw, pb):
    wspec = lambda shp: pl.BlockSpec(shp, lambda i: (0, 0))
    return pl.pallas_call(
        _tc_readout_body,
        grid=(N_PAD // BN,),
        in_specs=[pl.BlockSpec((BN, H), lambda i: (i, 0)),
                  wspec((1, H)), wspec((H, H)), wspec((1, H)),
                  pl.BlockSpec((BN, 1), lambda i: (i, 0)),
                  wspec((H, 1)), wspec((1, 1))],
        out_specs=pl.BlockSpec((NGRAPH, 1), lambda i: (0, 0)),
        out_shape=jax.ShapeDtypeStruct((NGRAPH, 1), jnp.float32),
        scratch_shapes=[pltpu.VMEM((NGRAPH, H), jnp.float32)],
    )(h2p, fl1b, fl2w, fl2b, batch2d, pw, pb)


def kernel(z, pos, batch, edge_index, emb, mw1_0, mb1_0, mw2_0, mb2_0, l1w_0,
           l2w_0, l2b_0, lw_0, lb_0, mw1_1, mb1_1, mw2_1, mb2_1, l1w_1,
           l2w_1, l2b_1, lw_1, lb_1, fl1w, fl1b, fl2w, fl2b, pw, pb):
    src = edge_index[0].astype(jnp.int32)
    dst = edge_index[1].astype(jnp.int32)
    epad = E_PAD - E
    src_p = jnp.concatenate([src, jnp.zeros((epad,), jnp.int32)])
    dst_p = jnp.concatenate([dst, jnp.zeros((epad,), jnp.int32)])
    z_p = jnp.concatenate([z.astype(jnp.int32),
                           jnp.zeros((N_PAD - N,), jnp.int32)])
    batch_p = jnp.concatenate([batch.astype(jnp.int32),
                               jnp.full((N_PAD - N,), NGRAPH, jnp.int32)])
    zeros_tbl = jnp.zeros((N_PAD, H), jnp.float32)

    mw1p0 = jnp.zeros((16, H), jnp.float32).at[:NGAUSS].set(mw1_0)
    mw1p1 = jnp.zeros((16, H), jnp.float32).at[:NGAUSS].set(mw1_1)
    fl1wp = jnp.zeros((H, H), jnp.float32).at[:, :H // 2].set(fl1w)
    fl1bp = jnp.zeros((1, H), jnp.float32).at[0, :H // 2].set(fl1b)
    fl2wp = jnp.zeros((H, H), jnp.float32).at[:H // 2].set(fl2w)
    r2 = lambda v: v.reshape(1, -1)

    dist2, h0 = _sc_prep(pos[:, 0], pos[:, 1], pos[:, 2], src_p, dst_p,
                         z_p, emb)
    dist_pk, c_pk = _tc_cdist(dist2.reshape(EROWS, H))
    wf0 = _tc_wf(dist_pk, c_pk, mw1p0, r2(mb1_0), mw2_0, r2(mb2_0))

    xl0 = _tc_xl(h0, l1w_0)
    agg0 = _sc_msgpass(xl0, wf0, src_p, dst_p, zeros_tbl)
    wf1 = _tc_wf(dist_pk, c_pk, mw1p1, r2(mb1_1), mw2_1, r2(mb2_1))
    h1, xl1 = _tc_update(agg0[0], agg0[1], h0, l2w_0, r2(l2b_0), lw_0,
                         r2(lb_0), l1w_1)

    agg1 = _sc_msgpass(xl1, wf1, src_p, dst_p, zeros_tbl)
    h2, h2p = _tc_update(agg1[0], agg1[1], h1, l2w_1, r2(l2b_1), lw_1,
                         r2(lb_1), fl1wp)
    del h2

    out = _tc_readout(h2p, fl1bp, fl2wp, r2(fl2b), batch_p.reshape(N_PAD, 1),
                      pw, r2(pb))
    return out
